# Initial kernel scaffold; baseline (speedup 1.0000x reference)
import functools

import jax
import jax.numpy as jnp
from jax import lax
from jax.experimental import pallas as pl
from jax.experimental.pallas import tpu as pltpu
from jax.experimental.pallas import tpu_sc as plsc

NC, NS, LANES = 2, 16, 16
NW = NC * NS
BLK = 80
ROWW = 160


def _lr(x):
    return jnp.maximum(x, 0.2 * x)


def _lane_bcast(v, lane):
    idx = jnp.full((LANES, 1), lane, jnp.int32)
    dn = lax.GatherDimensionNumbers(
        offset_dims=(), collapsed_slice_dims=(0,), start_index_map=(0,))
    return lax.gather(v, idx, dn, (1,),
                      mode=lax.GatherScatterMode.PROMISE_IN_BOUNDS)


def _tc_dense1(h_p, P1, P2, Wc1, Np):
    def body(h_ref, p1_ref, p2_ref, w1_ref, a_ref, b_ref, u_ref):
        hb = h_ref[...]
        a_ref[...] = hb @ p1_ref[...]
        b_ref[...] = hb @ p2_ref[...]
        u_ref[...] = hb @ w1_ref[...]

    g = Np // 256
    wspec = pl.BlockSpec((128, 128), lambda i: (0, 0))
    nspec = pl.BlockSpec((256, 128), lambda i: (i, 0))
    return pl.pallas_call(
        body,
        grid=(g,),
        in_specs=[nspec, wspec, wspec, wspec],
        out_specs=[nspec, nspec, nspec],
        out_shape=[jax.ShapeDtypeStruct((Np, 128), jnp.float32)] * 3,
    )(h_p, P1, P2, Wc1)


def _tc_reltables(z, P3, M1, M2, R):
    def body(z_ref, p3_ref, m1_ref, m2_ref, c_ref, vz_ref):
        zz = z_ref[...]
        c_ref[...] = zz @ p3_ref[...]
        vz_ref[...] = zz @ m1_ref[...] + m2_ref[...]

    return pl.pallas_call(
        body,
        in_specs=[pl.BlockSpec((R, 16), lambda: (0, 0)),
                  pl.BlockSpec((16, 128), lambda: (0, 0)),
                  pl.BlockSpec((16, ROWW), lambda: (0, 0)),
                  pl.BlockSpec((R, ROWW), lambda: (0, 0))],
        out_specs=[pl.BlockSpec((R, 128), lambda: (0, 0)),
                   pl.BlockSpec((R, ROWW), lambda: (0, 0))],
        out_shape=[jax.ShapeDtypeStruct((R, 128), jnp.float32),
                   jax.ShapeDtypeStruct((R, ROWW), jnp.float32)],
    )(z, P3, M1, M2)


def _sc_edge_pre(A, B, C, src, dst, et, E):
    epw = E // NW
    nblk = epw // BLK
    mesh = plsc.VectorSubcoreMesh(core_axis_name="c", subcore_axis_name="s")

    @functools.partial(
        pl.kernel, mesh=mesh,
        out_type=jax.ShapeDtypeStruct((E, 128), jnp.float32),
        scratch_types=[pltpu.VMEM((BLK,), jnp.int32)] * 3
        + [pltpu.VMEM((BLK, 128), jnp.float32)] * 4
        + [pltpu.SemaphoreType.DMA],
    )
    def k(a_h, b_h, c_h, src_h, dst_h, et_h, pre_h,
          si, di, ei, ab, bb, cb, pb, sem):
        w = lax.axis_index("s") * NC + lax.axis_index("c")
        base0 = pl.multiple_of(w * epw, 8)

        def blk(bi, carry):
            base = pl.multiple_of(base0 + bi * BLK, 8)
            pltpu.sync_copy(src_h.at[pl.ds(base, BLK)], si)
            pltpu.sync_copy(dst_h.at[pl.ds(base, BLK)], di)
            pltpu.sync_copy(et_h.at[pl.ds(base, BLK)], ei)
            g1 = pltpu.async_copy(a_h.at[di], ab, sem)
            g2 = pltpu.async_copy(b_h.at[si], bb, sem)
            g3 = pltpu.async_copy(c_h.at[ei], cb, sem)
            g1.wait()
            g2.wait()
            g3.wait()

            def edge(e, c2):
                for c in range(8):
                    s = pl.ds(c * 16, 16)
                    pb[e, s] = ab[e, s] + bb[e, s] + cb[e, s]
                return c2

            lax.fori_loop(0, BLK, edge, 0)
            pltpu.sync_copy(pb, pre_h.at[pl.ds(base, BLK)])
            return carry

        lax.fori_loop(0, nblk, blk, 0)

    return k(A, B, C, src, dst, et)


def _tc_edge_exp(pre, y_hat, E):
    tb = 1000

    def body(pre_ref, y_ref, out_ref):
        s = _lr(pre_ref[...]) @ y_ref[...]
        e = jnp.exp(s)
        out_ref[...] = jnp.concatenate([e, jnp.zeros_like(e)], axis=1)

    return pl.pallas_call(
        body,
        grid=(E // tb,),
        in_specs=[pl.BlockSpec((tb, 128), lambda i: (i, 0)),
                  pl.BlockSpec((128, 8), lambda i: (0, 0))],
        out_specs=pl.BlockSpec((tb, 16), lambda i: (i, 0)),
        out_shape=jax.ShapeDtypeStruct((E, 16), jnp.float32),
    )(pre, y_hat)


def _sc_edge_agg(U, VZ, eE, src, dst, et, E, Np):
    epw = E // NW
    nblk = epw // BLK
    mesh = plsc.VectorSubcoreMesh(core_axis_name="c", subcore_axis_name="s")

    @functools.partial(
        pl.kernel, mesh=mesh,
        out_type=jax.ShapeDtypeStruct((NC, Np, ROWW), jnp.float32),
        scratch_types=[pltpu.VMEM((BLK,), jnp.int32)] * 3
        + [pltpu.VMEM((BLK, 128), jnp.float32),
           pltpu.VMEM((BLK, ROWW), jnp.float32),
           pltpu.VMEM((BLK, 16), jnp.float32),
           pltpu.VMEM((BLK, ROWW), jnp.float32),
           pltpu.VMEM_SHARED((Np, ROWW), jnp.float32),
           pltpu.SemaphoreType.DMA],
    )
    def k(u_h, vz_h, e_h, src_h, dst_h, et_h, out_h,
          si, di, ei, ub, vzb, eb, mb, acc, sem):
        cid = lax.axis_index("c")
        sid = lax.axis_index("s")
        base0 = pl.multiple_of((sid * NC + cid) * epw, 8)
        nz = ROWW // 16

        def zrow(i, c2):
            e = i // nz
            c = i % nz
            mb[e, pl.ds(c * 16, 16)] = jnp.zeros((16,), jnp.float32)
            return c2

        lax.fori_loop(0, BLK * nz, zrow, 0)
        rps = Np // NS
        arow0 = sid * rps
        for kk in range(rps // BLK):
            pltpu.sync_copy(mb, acc.at[pl.ds(arow0 + kk * BLK, BLK)])
        plsc.subcore_barrier()

        def blk(bi, carry):
            base = pl.multiple_of(base0 + bi * BLK, 8)
            pltpu.sync_copy(src_h.at[pl.ds(base, BLK)], si)
            pltpu.sync_copy(dst_h.at[pl.ds(base, BLK)], di)
            pltpu.sync_copy(et_h.at[pl.ds(base, BLK)], ei)
            pltpu.sync_copy(e_h.at[pl.ds(base, BLK)], eb)
            g1 = pltpu.async_copy(u_h.at[si], ub, sem)
            g2 = pltpu.async_copy(vz_h.at[ei], vzb, sem)
            g1.wait()
            g2.wait()

            def edge(e, c2):
                er = eb[e, :]
                for c in range(8):
                    s = pl.ds(c * 16, 16)
                    mb[e, s] = _lane_bcast(er, c) * (ub[e, s] + vzb[e, s])
                mb[e, pl.ds(128, 16)] = er + vzb[e, pl.ds(128, 16)]
                mb[e, pl.ds(144, 16)] = vzb[e, pl.ds(144, 16)]
                return c2

            lax.fori_loop(0, BLK, edge, 0)
            pltpu.sync_copy(mb, acc.at[di], add=True)
            return carry

        lax.fori_loop(0, nblk, blk, 0)
        plsc.subcore_barrier()
        drow = sid * rps
        pltpu.sync_copy(acc.at[pl.ds(drow, rps)],
                        out_h.at[cid, pl.ds(drow, rps)])

    return k(U, VZ, eE, src, dst, et)


def _tc_final(A, B, U, h_p, parts, P3, Wc2, y_hat, E8, res2, Np):
    m0 = parts[0]
    m1 = parts[1]

    def body(a_ref, b_ref, u_ref, h_ref, m0_ref, m1_ref,
             p3_ref, w2_ref, y_ref, e8_ref, r_ref, out_ref):
        m = m0_ref[...] + m1_ref[...]
        deg = m[:, 136:137]
        zbar = m[:, 137:153] / jnp.maximum(deg, 1.0)
        pre = a_ref[...] + b_ref[...] + zbar @ p3_ref[...]
        es = jnp.exp(_lr(pre) @ y_ref[...])
        ts = u_ref[...] + zbar @ w2_ref[...]
        den = es + m[:, 128:136]
        e128 = es @ e8_ref[...]
        d128 = den @ e8_ref[...]
        agg = (e128 * ts + m[:, 0:128]) / d128
        hh = h_ref[...]
        hn = _lr(agg + r_ref[0, 0] * hh)
        out_ref[...] = jnp.where(deg > 0.0, hn, hh)

    g = Np // 256
    nspec = pl.BlockSpec((256, 128), lambda i: (i, 0))
    mspec = pl.BlockSpec((256, ROWW), lambda i: (i, 0))
    return pl.pallas_call(
        body,
        grid=(g,),
        in_specs=[nspec, nspec, nspec, nspec, mspec, mspec,
                  pl.BlockSpec((16, 128), lambda i: (0, 0)),
                  pl.BlockSpec((16, 128), lambda i: (0, 0)),
                  pl.BlockSpec((128, 8), lambda i: (0, 0)),
                  pl.BlockSpec((8, 128), lambda i: (0, 0)),
                  pl.BlockSpec(memory_space=pltpu.SMEM)],
        out_specs=nspec,
        out_shape=jax.ShapeDtypeStruct((Np, 128), jnp.float32),
    )(A, B, U, h_p, m0, m1, P3, Wc2, y_hat, E8, res2)


def kernel(h, z, edge_index, edge_type, Wc_w, P_hat_w, y_hat_w, res_w):
    N, D = h.shape
    R, RD = z.shape
    E = edge_type.shape[0]
    Np = ((N + 255) // 256) * 256

    src = edge_index[0].astype(jnp.int32)
    dst = edge_index[1].astype(jnp.int32)
    et = edge_type.astype(jnp.int32)

    P1 = P_hat_w[:D]
    P2 = P_hat_w[D:2 * D]
    P3 = P_hat_w[2 * D:]
    Wc1 = Wc_w[:D]
    Wc2 = Wc_w[D:]

    M1 = jnp.concatenate(
        [Wc2, jnp.zeros((RD, 9), jnp.float32), jnp.eye(RD, dtype=jnp.float32),
         jnp.zeros((RD, ROWW - 153), jnp.float32)], axis=1)
    M2 = jnp.broadcast_to(
        (jnp.arange(ROWW) == 136).astype(jnp.float32)[None, :], (R, ROWW))
    E8 = jnp.kron(jnp.eye(8, dtype=jnp.float32),
                  jnp.ones((1, 16), jnp.float32))
    res2 = res_w.reshape(1, 1)

    h_p = jnp.pad(h, ((0, Np - N), (0, 0)))

    A, B, U = _tc_dense1(h_p, P1, P2, Wc1, Np)
    C, VZ = _tc_reltables(z, P3, M1, M2, R)
    pre = _sc_edge_pre(A, B, C, src, dst, et, E)
    eE = _tc_edge_exp(pre, y_hat_w, E)
    parts = _sc_edge_agg(U, VZ, eE, src, dst, et, E, Np)
    out = _tc_final(A, B, U, h_p, parts, P3, Wc2, y_hat_w, E8, res2, Np)
    return out[:N]


# speedup vs baseline: 18.2861x; 18.2861x over previous
import functools

import jax
import jax.numpy as jnp
from jax import lax
from jax.experimental import pallas as pl
from jax.experimental.pallas import tpu as pltpu
from jax.experimental.pallas import tpu_sc as plsc

NC, NS, LANES = 2, 16, 16
NW = NC * NS
BLK = 80
ROWW = 144
AUXW = 32
_SC_PARAMS = pltpu.CompilerParams(use_tc_tiling_on_sc=False)


def _lr(x):
    return jnp.maximum(x, 0.2 * x)


def _lane_bcast(v, lane):
    idx = jnp.full((LANES, 1), lane, jnp.int32)
    dn = lax.GatherDimensionNumbers(
        offset_dims=(), collapsed_slice_dims=(0,), start_index_map=(0,))
    return lax.gather(v, idx, dn, (1,),
                      mode=lax.GatherScatterMode.PROMISE_IN_BOUNDS)


def _tc_dense1(h_p, P1, P2, Wc1, Np):
    def body(h_ref, p1_ref, p2_ref, w1_ref, a_ref, b_ref, u_ref):
        hb = h_ref[...]
        a_ref[...] = hb @ p1_ref[...]
        b_ref[...] = hb @ p2_ref[...]
        u_ref[...] = hb @ w1_ref[...]

    g = Np // 256
    wspec = pl.BlockSpec((128, 128), lambda i: (0, 0))
    nspec = pl.BlockSpec((256, 128), lambda i: (i, 0))
    return pl.pallas_call(
        body,
        grid=(g,),
        in_specs=[nspec, wspec, wspec, wspec],
        out_specs=[nspec, nspec, nspec],
        out_shape=[jax.ShapeDtypeStruct((Np, 128), jnp.float32)] * 3,
    )(h_p, P1, P2, Wc1)


def _tc_reltables(z, P3, M1, M3, ONE0, R):
    def body(z_ref, p3_ref, m1_ref, m3_ref, o_ref, c_ref, v_ref, tz_ref):
        zz = z_ref[...]
        c_ref[...] = zz @ p3_ref[...]
        v_ref[...] = zz @ m1_ref[...]
        tz_ref[...] = zz @ m3_ref[...] + o_ref[...]

    return pl.pallas_call(
        body,
        in_specs=[pl.BlockSpec((R, 16), lambda: (0, 0)),
                  pl.BlockSpec((16, 128), lambda: (0, 0)),
                  pl.BlockSpec((16, ROWW), lambda: (0, 0)),
                  pl.BlockSpec((16, AUXW), lambda: (0, 0)),
                  pl.BlockSpec((R, AUXW), lambda: (0, 0))],
        out_specs=[pl.BlockSpec((R, 128), lambda: (0, 0)),
                   pl.BlockSpec((R, ROWW), lambda: (0, 0)),
                   pl.BlockSpec((R, AUXW), lambda: (0, 0))],
        out_shape=[jax.ShapeDtypeStruct((R, 128), jnp.float32),
                   jax.ShapeDtypeStruct((R, ROWW), jnp.float32),
                   jax.ShapeDtypeStruct((R, AUXW), jnp.float32)],
    )(z, P3, M1, M3, ONE0)


def _sc_edge_pre(A, B, C, TZ, src, dst, et, E, Np):
    epw = E // NW
    nblk = epw // BLK
    mesh = plsc.VectorSubcoreMesh(core_axis_name="c", subcore_axis_name="s")

    @functools.partial(
        pl.kernel, mesh=mesh,
        out_type=[jax.ShapeDtypeStruct((E, 128), jnp.float32),
                  jax.ShapeDtypeStruct((NC, Np, AUXW), jnp.float32)],
        compiler_params=_SC_PARAMS,
        scratch_types=[pltpu.VMEM((BLK,), jnp.int32)] * 3
        + [pltpu.VMEM((BLK, 128), jnp.float32)] * 4
        + [pltpu.VMEM((BLK, AUXW), jnp.float32),
           pltpu.VMEM_SHARED((Np, AUXW), jnp.float32),
           pltpu.SemaphoreType.DMA],
    )
    def k(a_h, b_h, c_h, tz_h, src_h, dst_h, et_h, pre_h, aux_h,
          si, di, ei, ab, bb, cb, pb, tb, acc, sem):
        cid = lax.axis_index("c")
        sid = lax.axis_index("s")
        base0 = pl.multiple_of((sid * NC + cid) * epw, 8)

        def zrow(i, c2):
            tb[i // 2, pl.ds((i % 2) * 16, 16)] = jnp.zeros((16,), jnp.float32)
            return c2

        lax.fori_loop(0, BLK * (AUXW // 16), zrow, 0)
        rps = Np // NS
        for kk in range(rps // BLK):
            pltpu.sync_copy(tb, acc.at[pl.ds(sid * rps + kk * BLK, BLK)])
        plsc.subcore_barrier()

        def blk(bi, carry):
            base = pl.multiple_of(base0 + bi * BLK, 8)
            pltpu.sync_copy(src_h.at[pl.ds(base, BLK)], si)
            pltpu.sync_copy(dst_h.at[pl.ds(base, BLK)], di)
            pltpu.sync_copy(et_h.at[pl.ds(base, BLK)], ei)
            g1 = pltpu.async_copy(a_h.at[di], ab, sem)
            g2 = pltpu.async_copy(b_h.at[si], bb, sem)
            g3 = pltpu.async_copy(c_h.at[ei], cb, sem)
            g4 = pltpu.async_copy(tz_h.at[ei], tb, sem)
            g1.wait()
            g2.wait()
            g3.wait()
            g4.wait()

            def edge(e, c2):
                for c in range(8):
                    s = pl.ds(c * 16, 16)
                    pb[e, s] = ab[e, s] + bb[e, s] + cb[e, s]
                return c2

            lax.fori_loop(0, BLK, edge, 0)
            pltpu.sync_copy(pb, pre_h.at[pl.ds(base, BLK)])
            pltpu.sync_copy(tb, acc.at[di], add=True)
            return carry

        lax.fori_loop(0, nblk, blk, 0)
        plsc.subcore_barrier()
        pltpu.sync_copy(acc.at[pl.ds(sid * rps, rps)],
                        aux_h.at[cid, pl.ds(sid * rps, rps)])

    return k(A, B, C, TZ, src, dst, et)


def _tc_edge_exp(pre, y_hat, E):
    tb = 1000

    def body(pre_ref, y_ref, out_ref):
        s = _lr(pre_ref[...]) @ y_ref[...]
        e = jnp.exp(s)
        out_ref[...] = jnp.concatenate([e, jnp.zeros_like(e)], axis=1)

    return pl.pallas_call(
        body,
        grid=(E // tb,),
        in_specs=[pl.BlockSpec((tb, 128), lambda i: (i, 0)),
                  pl.BlockSpec((128, 8), lambda i: (0, 0))],
        out_specs=pl.BlockSpec((tb, 16), lambda i: (i, 0)),
        out_shape=jax.ShapeDtypeStruct((E, 16), jnp.float32),
    )(pre, y_hat)


def _sc_edge_agg(U, V, eE, src, dst, et, E, Np):
    epw = E // NW
    nblk = epw // BLK
    mesh = plsc.VectorSubcoreMesh(core_axis_name="c", subcore_axis_name="s")

    @functools.partial(
        pl.kernel, mesh=mesh,
        out_type=jax.ShapeDtypeStruct((NC, Np, ROWW), jnp.float32),
        compiler_params=_SC_PARAMS,
        scratch_types=[pltpu.VMEM((BLK,), jnp.int32)] * 3
        + [pltpu.VMEM((BLK, 128), jnp.float32),
           pltpu.VMEM((BLK, ROWW), jnp.float32),
           pltpu.VMEM((BLK, 16), jnp.float32),
           pltpu.VMEM((BLK, ROWW), jnp.float32),
           pltpu.VMEM_SHARED((Np, ROWW), jnp.float32),
           pltpu.SemaphoreType.DMA],
    )
    def k(u_h, v_h, e_h, src_h, dst_h, et_h, out_h,
          si, di, ei, ub, vb, eb, mb, acc, sem):
        cid = lax.axis_index("c")
        sid = lax.axis_index("s")
        base0 = pl.multiple_of((sid * NC + cid) * epw, 8)
        nz = ROWW // 16

        def zrow(i, c2):
            mb[i // nz, pl.ds((i % nz) * 16, 16)] = jnp.zeros((16,),
                                                              jnp.float32)
            return c2

        lax.fori_loop(0, BLK * nz, zrow, 0)
        rps = Np // NS
        for kk in range(rps // BLK):
            pltpu.sync_copy(mb, acc.at[pl.ds(sid * rps + kk * BLK, BLK)])
        plsc.subcore_barrier()

        def blk(bi, carry):
            base = pl.multiple_of(base0 + bi * BLK, 8)
            pltpu.sync_copy(src_h.at[pl.ds(base, BLK)], si)
            pltpu.sync_copy(dst_h.at[pl.ds(base, BLK)], di)
            pltpu.sync_copy(et_h.at[pl.ds(base, BLK)], ei)
            pltpu.sync_copy(e_h.at[pl.ds(base, BLK)], eb)
            g1 = pltpu.async_copy(u_h.at[si], ub, sem)
            g2 = pltpu.async_copy(v_h.at[ei], vb, sem)
            g1.wait()
            g2.wait()

            def edge(e, c2):
                er = eb[e, :]
                for c in range(8):
                    s = pl.ds(c * 16, 16)
                    mb[e, s] = _lane_bcast(er, c) * (ub[e, s] + vb[e, s])
                mb[e, pl.ds(128, 16)] = er
                return c2

            lax.fori_loop(0, BLK, edge, 0)
            pltpu.sync_copy(mb, acc.at[di], add=True)
            return carry

        lax.fori_loop(0, nblk, blk, 0)
        plsc.subcore_barrier()
        pltpu.sync_copy(acc.at[pl.ds(sid * rps, rps)],
                        out_h.at[cid, pl.ds(sid * rps, rps)])

    return k(U, V, eE, src, dst, et)


def _tc_final(A, B, U, h_p, parts, aux, P3, Wc2, y_hat, E8, res2, Np):
    def body(a_ref, b_ref, u_ref, h_ref, m0_ref, m1_ref, x0_ref, x1_ref,
             p3_ref, w2_ref, y_ref, e8_ref, r_ref, out_ref):
        x = x0_ref[...] + x1_ref[...]
        deg = x[:, 0:1]
        zbar = x[:, 1:17] / jnp.maximum(deg, 1.0)
        m = m0_ref[...] + m1_ref[...]
        pre = a_ref[...] + b_ref[...] + zbar @ p3_ref[...]
        es = jnp.exp(_lr(pre) @ y_ref[...])
        ts = u_ref[...] + zbar @ w2_ref[...]
        den = es + m[:, 128:136]
        e128 = es @ e8_ref[...]
        d128 = den @ e8_ref[...]
        agg = (e128 * ts + m[:, 0:128]) / d128
        hh = h_ref[...]
        hn = _lr(agg + r_ref[0, 0] * hh)
        out_ref[...] = jnp.where(deg > 0.0, hn, hh)

    g = Np // 256
    nspec = pl.BlockSpec((256, 128), lambda i: (i, 0))
    mspec = pl.BlockSpec((256, ROWW), lambda i: (i, 0))
    xspec = pl.BlockSpec((256, AUXW), lambda i: (i, 0))
    return pl.pallas_call(
        body,
        grid=(g,),
        in_specs=[nspec, nspec, nspec, nspec, mspec, mspec, xspec, xspec,
                  pl.BlockSpec((16, 128), lambda i: (0, 0)),
                  pl.BlockSpec((16, 128), lambda i: (0, 0)),
                  pl.BlockSpec((128, 8), lambda i: (0, 0)),
                  pl.BlockSpec((8, 128), lambda i: (0, 0)),
                  pl.BlockSpec(memory_space=pltpu.SMEM)],
        out_specs=nspec,
        out_shape=jax.ShapeDtypeStruct((Np, 128), jnp.float32),
    )(A, B, U, h_p, parts[0], parts[1], aux[0], aux[1],
      P3, Wc2, y_hat, E8, res2)


def kernel(h, z, edge_index, edge_type, Wc_w, P_hat_w, y_hat_w, res_w):
    N, D = h.shape
    R, RD = z.shape
    E = edge_type.shape[0]
    Np = ((N + 255) // 256) * 256

    src = edge_index[0].astype(jnp.int32)
    dst = edge_index[1].astype(jnp.int32)
    et = edge_type.astype(jnp.int32)

    P1 = P_hat_w[:D]
    P2 = P_hat_w[D:2 * D]
    P3 = P_hat_w[2 * D:]
    Wc1 = Wc_w[:D]
    Wc2 = Wc_w[D:]

    M1 = jnp.concatenate([Wc2, jnp.zeros((RD, ROWW - 128), jnp.float32)],
                         axis=1)
    M3 = jnp.concatenate([jnp.zeros((RD, 1), jnp.float32),
                          jnp.eye(RD, dtype=jnp.float32),
                          jnp.zeros((RD, AUXW - RD - 1), jnp.float32)],
                         axis=1)
    ONE0 = jnp.broadcast_to(
        (jnp.arange(AUXW) == 0).astype(jnp.float32)[None, :], (R, AUXW))
    E8 = jnp.kron(jnp.eye(8, dtype=jnp.float32),
                  jnp.ones((1, 16), jnp.float32))
    res2 = res_w.reshape(1, 1)

    h_p = jnp.pad(h, ((0, Np - N), (0, 0)))

    A, B, U = _tc_dense1(h_p, P1, P2, Wc1, Np)
    C, V, TZ = _tc_reltables(z, P3, M1, M3, ONE0, R)
    pre, aux = _sc_edge_pre(A, B, C, TZ, src, dst, et, E, Np)
    eE = _tc_edge_exp(pre, y_hat_w, E)
    parts = _sc_edge_agg(U, V, eE, src, dst, et, E, Np)
    out = _tc_final(A, B, U, h_p, parts, aux, P3, Wc2, y_hat_w, E8, res2, Np)
    return out[:N]


# speedup vs baseline: 20.0853x; 1.0984x over previous
import functools

import jax
import jax.numpy as jnp
from jax import lax
from jax.experimental import pallas as pl
from jax.experimental.pallas import tpu as pltpu
from jax.experimental.pallas import tpu_sc as plsc

NC, NS, LANES = 2, 16, 16
NW = NC * NS
BLK1 = 128
BLK2 = 64
ROWW = 144
AUXW = 32
_SC_PARAMS = pltpu.CompilerParams(use_tc_tiling_on_sc=False,
                                  needs_layout_passes=False)


def _cols(c):
    return lax.iota(jnp.int32, 16) + (c * 16)


def _lr(x):
    return jnp.maximum(x, 0.2 * x)


def _lane_bcast(v, lane):
    idx = jnp.full((LANES, 1), lane, jnp.int32)
    dn = lax.GatherDimensionNumbers(
        offset_dims=(), collapsed_slice_dims=(0,), start_index_map=(0,))
    return lax.gather(v, idx, dn, (1,),
                      mode=lax.GatherScatterMode.PROMISE_IN_BOUNDS)


def _tc_dense1(h_p, P1, P2, Wc1, Np):
    def body(h_ref, p1_ref, p2_ref, w1_ref, a_ref, b_ref, u_ref):
        hb = h_ref[...]
        a_ref[...] = hb @ p1_ref[...]
        b_ref[...] = hb @ p2_ref[...]
        u_ref[...] = hb @ w1_ref[...]

    g = Np // 256
    wspec = pl.BlockSpec((128, 128), lambda i: (0, 0))
    nspec = pl.BlockSpec((256, 128), lambda i: (i, 0))
    return pl.pallas_call(
        body,
        grid=(g,),
        in_specs=[nspec, wspec, wspec, wspec],
        out_specs=[nspec, nspec, nspec],
        out_shape=[jax.ShapeDtypeStruct((Np, 128), jnp.float32)] * 3,
    )(h_p, P1, P2, Wc1)


def _tc_reltables(z, P3, Wc2, M3, ONE0, R):
    def body(z_ref, p3_ref, w2_ref, m3_ref, o_ref, c_ref, v_ref, tz_ref):
        zz = z_ref[...]
        c_ref[...] = zz @ p3_ref[...]
        v_ref[...] = zz @ w2_ref[...]
        tz_ref[...] = zz @ m3_ref[...] + o_ref[...]

    return pl.pallas_call(
        body,
        in_specs=[pl.BlockSpec((R, 16), lambda: (0, 0)),
                  pl.BlockSpec((16, 128), lambda: (0, 0)),
                  pl.BlockSpec((16, 128), lambda: (0, 0)),
                  pl.BlockSpec((16, AUXW), lambda: (0, 0)),
                  pl.BlockSpec((R, AUXW), lambda: (0, 0))],
        out_specs=[pl.BlockSpec((R, 128), lambda: (0, 0)),
                   pl.BlockSpec((R, 128), lambda: (0, 0)),
                   pl.BlockSpec((R, AUXW), lambda: (0, 0))],
        out_shape=[jax.ShapeDtypeStruct((R, 128), jnp.float32),
                   jax.ShapeDtypeStruct((R, 128), jnp.float32),
                   jax.ShapeDtypeStruct((R, AUXW), jnp.float32)],
    )(z, P3, Wc2, M3, ONE0)


def _sc_edge_pre(A, B, C, TZ, idx3, Ep, Np):
    epw = Ep // NW
    nblk = epw // BLK1
    nh = nblk // 2
    mesh = plsc.VectorSubcoreMesh(core_axis_name="c", subcore_axis_name="s")

    @functools.partial(
        pl.kernel, mesh=mesh,
        out_type=[jax.ShapeDtypeStruct((Ep, 128), jnp.float32),
                  jax.ShapeDtypeStruct((NC, Np, AUXW), jnp.float32)],
        compiler_params=_SC_PARAMS,
        scratch_types=[pltpu.VMEM((3, BLK1), jnp.int32)] * 2
        + [pltpu.VMEM((BLK1, 128), jnp.float32)] * 4
        + [pltpu.VMEM((BLK1, 128), jnp.float32),
           pltpu.VMEM((BLK1, AUXW), jnp.float32),
           pltpu.VMEM((64, 128), jnp.float32),
           pltpu.VMEM((64, AUXW), jnp.float32),
           pltpu.VMEM_SHARED((Np, AUXW), jnp.float32),
           pltpu.SemaphoreType.DMA,
           pltpu.SemaphoreType.DMA],
    )
    def k(a_h, b_h, c_h, tz_h, idx_h, pre_h, aux_h,
          ib0, ib1, ab0, bb0, ab1, bb1, pb, tb, cc, tzc, acc, sem0, sem1):
        cid = lax.axis_index("c")
        sid = lax.axis_index("s")
        w = sid * NC + cid
        base0 = pl.multiple_of(w * epw, 8)

        pltpu.sync_copy(c_h, cc)
        pltpu.sync_copy(tz_h, tzc)

        def zrow(i, c2):
            tb[i // 2, pl.ds((i % 2) * 16, 16)] = jnp.zeros((16,),
                                                            jnp.float32)
            return c2

        lax.fori_loop(0, BLK1 * (AUXW // 16), zrow, 0)
        rps = Np // NS
        for kk in range(rps // BLK1):
            pltpu.sync_copy(tb, acc.at[pl.ds(sid * rps + kk * BLK1, BLK1)])
        plsc.subcore_barrier()

        def fire(bi, ib, ab, bb, sem):
            pltpu.sync_copy(idx_h.at[w, bi], ib)
            pltpu.async_copy(a_h.at[ib.at[1]], ab, sem)
            pltpu.async_copy(b_h.at[ib.at[0]], bb, sem)

        def consume(bi, ib, ab, bb, sem):
            pltpu.make_async_copy(a_h.at[ib.at[1]], ab, sem).wait()
            pltpu.make_async_copy(b_h.at[ib.at[0]], bb, sem).wait()

            def edge(e, c2):
                etv = ib[2, pl.ds((e // 16) * 16, 16)]
                etb = _lane_bcast(etv, e % 16)
                for c in range(8):
                    s = pl.ds(c * 16, 16)
                    crow = plsc.load_gather(cc, [etb, _cols(c)])
                    pb[e, s] = ab[e, s] + bb[e, s] + crow
                tb[e, pl.ds(0, 16)] = plsc.load_gather(tzc, [etb, _cols(0)])
                tb[e, pl.ds(16, 16)] = plsc.load_gather(tzc, [etb, _cols(1)])
                return c2

            lax.fori_loop(0, BLK1, edge, 0)
            pltpu.sync_copy(pb, pre_h.at[pl.ds(base0 + bi * BLK1, BLK1)])
            pltpu.sync_copy(tb, acc.at[ib.at[1]], add=True)

        fire(0, ib0, ab0, bb0, sem0)

        def body(g, carry):
            fire(2 * g + 1, ib1, ab1, bb1, sem1)
            consume(2 * g, ib0, ab0, bb0, sem0)

            @pl.when(g < nh - 1)
            def _():
                fire(2 * g + 2, ib0, ab0, bb0, sem0)

            consume(2 * g + 1, ib1, ab1, bb1, sem1)
            return carry

        lax.fori_loop(0, nh, body, 0)
        plsc.subcore_barrier()
        pltpu.sync_copy(acc.at[pl.ds(sid * rps, rps)],
                        aux_h.at[cid, pl.ds(sid * rps, rps)])

    return k(A, B, C, TZ, idx3)


def _tc_edge_exp(pre, y_hat, Ep):
    tb = 1024

    def body(pre_ref, y_ref, out_ref):
        s = _lr(pre_ref[...]) @ y_ref[...]
        e = jnp.exp(s)
        out_ref[...] = jnp.concatenate([e, jnp.zeros_like(e)], axis=1)

    return pl.pallas_call(
        body,
        grid=(Ep // tb,),
        in_specs=[pl.BlockSpec((tb, 128), lambda i: (i, 0)),
                  pl.BlockSpec((128, 8), lambda i: (0, 0))],
        out_specs=pl.BlockSpec((tb, 16), lambda i: (i, 0)),
        out_shape=jax.ShapeDtypeStruct((Ep, 16), jnp.float32),
    )(pre, y_hat)


def _sc_edge_agg(U, V, eE4, idx3, Ep, Np):
    epw = Ep // NW
    nblk = epw // BLK2
    nh = nblk // 2
    mesh = plsc.VectorSubcoreMesh(core_axis_name="c", subcore_axis_name="s")

    @functools.partial(
        pl.kernel, mesh=mesh,
        out_type=jax.ShapeDtypeStruct((NC, Np, ROWW), jnp.float32),
        compiler_params=_SC_PARAMS,
        scratch_types=[pltpu.VMEM((3, BLK2), jnp.int32)] * 2
        + [pltpu.VMEM((BLK2, 128), jnp.float32),
           pltpu.VMEM((BLK2, 16), jnp.float32)] * 2
        + [pltpu.VMEM((BLK2, ROWW), jnp.float32),
           pltpu.VMEM((64, 128), jnp.float32),
           pltpu.VMEM_SHARED((Np, ROWW), jnp.float32),
           pltpu.SemaphoreType.DMA,
           pltpu.SemaphoreType.DMA],
    )
    def k(u_h, v_h, e_h, idx_h, out_h,
          ib0, ib1, ub0, eb0, ub1, eb1, mb, vc, acc, sem0, sem1):
        cid = lax.axis_index("c")
        sid = lax.axis_index("s")
        w = sid * NC + cid
        nz = ROWW // 16

        pltpu.sync_copy(v_h, vc)

        def zrow(i, c2):
            mb[i // nz, pl.ds((i % nz) * 16, 16)] = jnp.zeros((16,),
                                                              jnp.float32)
            return c2

        lax.fori_loop(0, BLK2 * nz, zrow, 0)
        rps = Np // NS
        for kk in range(rps // BLK2):
            pltpu.sync_copy(mb, acc.at[pl.ds(sid * rps + kk * BLK2, BLK2)])
        plsc.subcore_barrier()

        def fire(bi, ib, ub, eb, sem):
            pltpu.sync_copy(idx_h.at[w, bi], ib)
            pltpu.async_copy(u_h.at[ib.at[0]], ub, sem)
            pltpu.async_copy(e_h.at[w, bi], eb, sem)

        def consume(bi, ib, ub, eb, sem):
            pltpu.make_async_copy(u_h.at[ib.at[0]], ub, sem).wait()
            pltpu.make_async_copy(e_h.at[w, bi], eb, sem).wait()

            def edge(e, c2):
                er = eb[e, :]
                etv = ib[2, pl.ds((e // 16) * 16, 16)]
                etb = _lane_bcast(etv, e % 16)
                for c in range(8):
                    s = pl.ds(c * 16, 16)
                    vrow = plsc.load_gather(vc, [etb, _cols(c)])
                    mb[e, s] = _lane_bcast(er, c) * (ub[e, s] + vrow)
                mb[e, pl.ds(128, 16)] = er
                return c2

            lax.fori_loop(0, BLK2, edge, 0)
            pltpu.sync_copy(mb, acc.at[ib.at[1]], add=True)

        fire(0, ib0, ub0, eb0, sem0)

        def body(g, carry):
            fire(2 * g + 1, ib1, ub1, eb1, sem1)
            consume(2 * g, ib0, ub0, eb0, sem0)

            @pl.when(g < nh - 1)
            def _():
                fire(2 * g + 2, ib0, ub0, eb0, sem0)

            consume(2 * g + 1, ib1, ub1, eb1, sem1)
            return carry

        lax.fori_loop(0, nh, body, 0)
        plsc.subcore_barrier()
        pltpu.sync_copy(acc.at[pl.ds(sid * rps, rps)],
                        out_h.at[cid, pl.ds(sid * rps, rps)])

    return k(U, V, eE4, idx3)


def _tc_final(A, B, U, h_p, parts, aux, P3, Wc2, y_hat, E8, res2, Np):
    def body(a_ref, b_ref, u_ref, h_ref, m0_ref, m1_ref, x0_ref, x1_ref,
             p3_ref, w2_ref, y_ref, e8_ref, r_ref, out_ref):
        x = x0_ref[...] + x1_ref[...]
        deg = x[:, 0:1]
        zbar = x[:, 1:17] / jnp.maximum(deg, 1.0)
        m = m0_ref[...] + m1_ref[...]
        pre = a_ref[...] + b_ref[...] + zbar @ p3_ref[...]
        es = jnp.exp(_lr(pre) @ y_ref[...])
        ts = u_ref[...] + zbar @ w2_ref[...]
        den = es + m[:, 128:136]
        e128 = es @ e8_ref[...]
        d128 = den @ e8_ref[...]
        agg = (e128 * ts + m[:, 0:128]) / d128
        hh = h_ref[...]
        hn = _lr(agg + r_ref[0, 0] * hh)
        out_ref[...] = jnp.where(deg > 0.0, hn, hh)

    g = Np // 256
    nspec = pl.BlockSpec((256, 128), lambda i: (i, 0))
    mspec = pl.BlockSpec((256, ROWW), lambda i: (i, 0))
    xspec = pl.BlockSpec((256, AUXW), lambda i: (i, 0))
    return pl.pallas_call(
        body,
        grid=(g,),
        in_specs=[nspec, nspec, nspec, nspec, mspec, mspec, xspec, xspec,
                  pl.BlockSpec((16, 128), lambda i: (0, 0)),
                  pl.BlockSpec((16, 128), lambda i: (0, 0)),
                  pl.BlockSpec((128, 8), lambda i: (0, 0)),
                  pl.BlockSpec((8, 128), lambda i: (0, 0)),
                  pl.BlockSpec(memory_space=pltpu.SMEM)],
        out_specs=nspec,
        out_shape=jax.ShapeDtypeStruct((Np, 128), jnp.float32),
    )(A, B, U, h_p, parts[0], parts[1], aux[0], aux[1],
      P3, Wc2, y_hat, E8, res2)


def kernel(h, z, edge_index, edge_type, Wc_w, P_hat_w, y_hat_w, res_w):
    N, D = h.shape
    R, RD = z.shape
    E = edge_type.shape[0]
    Np = ((N + 255) // 256) * 256
    gran = NW * BLK1 * 2
    Ep = ((E + gran - 1) // gran) * gran
    epw = Ep // NW

    src = edge_index[0].astype(jnp.int32)
    dst = edge_index[1].astype(jnp.int32)
    et = edge_type.astype(jnp.int32)
    srcp = jnp.full((Ep,), Np - 1, jnp.int32).at[:E].set(src)
    dstp = jnp.full((Ep,), Np - 1, jnp.int32).at[:E].set(dst)
    etp = jnp.zeros((Ep,), jnp.int32).at[:E].set(et)
    sde = jnp.stack([srcp.reshape(NW, epw), dstp.reshape(NW, epw),
                     etp.reshape(NW, epw)], axis=1)
    idx1 = jnp.swapaxes(sde.reshape(NW, 3, epw // BLK1, BLK1), 1, 2)
    idx2 = jnp.swapaxes(sde.reshape(NW, 3, epw // BLK2, BLK2), 1, 2)

    P1 = P_hat_w[:D]
    P2 = P_hat_w[D:2 * D]
    P3 = P_hat_w[2 * D:]
    Wc1 = Wc_w[:D]
    Wc2 = Wc_w[D:]

    M3 = jnp.concatenate([jnp.zeros((RD, 1), jnp.float32),
                          jnp.eye(RD, dtype=jnp.float32),
                          jnp.zeros((RD, AUXW - RD - 1), jnp.float32)],
                         axis=1)
    ONE0 = jnp.broadcast_to(
        (jnp.arange(AUXW) == 0).astype(jnp.float32)[None, :], (R, AUXW))
    E8 = jnp.kron(jnp.eye(8, dtype=jnp.float32),
                  jnp.ones((1, 16), jnp.float32))
    res2 = res_w.reshape(1, 1)

    h_p = jnp.pad(h, ((0, Np - N), (0, 0)))

    A, B, U = _tc_dense1(h_p, P1, P2, Wc1, Np)
    C, V, TZ = _tc_reltables(z, P3, Wc2, M3, ONE0, R)
    pre, aux = _sc_edge_pre(A, B, C, TZ, idx1, Ep, Np)
    eE = _tc_edge_exp(pre, y_hat_w, Ep)
    eE4 = eE.reshape(NW, epw // BLK2, BLK2, 16)
    parts = _sc_edge_agg(U, V, eE4, idx2, Ep, Np)
    out = _tc_final(A, B, U, h_p, parts, aux, P3, Wc2, y_hat_w, E8, res2, Np)
    return out[:N]


# speedup vs baseline: 24.4745x; 1.2185x over previous
import functools

import jax
import jax.numpy as jnp
from jax import lax
from jax.experimental import pallas as pl
from jax.experimental.pallas import tpu as pltpu
from jax.experimental.pallas import tpu_sc as plsc

NC, NS, LANES = 2, 16, 16
NW = NC * NS
BLK1 = 128
BLK2 = 32
ROWW = 144
AUXW = 24
_SC_PARAMS = pltpu.CompilerParams(use_tc_tiling_on_sc=False,
                                  needs_layout_passes=False)


def _lr(x):
    return jnp.maximum(x, 0.2 * x)


def _lane_bcast(v, lane):
    idx = jnp.full((LANES, 1), lane, jnp.int32)
    dn = lax.GatherDimensionNumbers(
        offset_dims=(), collapsed_slice_dims=(0,), start_index_map=(0,))
    return lax.gather(v, idx, dn, (1,),
                      mode=lax.GatherScatterMode.PROMISE_IN_BOUNDS)


def _tc_dense1(h_p, P1, P2, Wc1, Np):
    def body(h_ref, p1_ref, p2_ref, w1_ref, a_ref, b_ref, u_ref):
        hb = h_ref[...]
        a_ref[...] = hb @ p1_ref[...]
        b_ref[...] = hb @ p2_ref[...]
        u_ref[...] = hb @ w1_ref[...]

    g = Np // 256
    wspec = pl.BlockSpec((128, 128), lambda i: (0, 0))
    nspec = pl.BlockSpec((256, 128), lambda i: (i, 0))
    return pl.pallas_call(
        body,
        grid=(g,),
        in_specs=[nspec, wspec, wspec, wspec],
        out_specs=[nspec, nspec, nspec],
        out_shape=[jax.ShapeDtypeStruct((Np, 128), jnp.float32)] * 3,
    )(h_p, P1, P2, Wc1)


def _tc_reltables(z, P3, Wc2, M3, ONE0, R):
    def body(z_ref, p3_ref, w2_ref, m3_ref, o_ref, c_ref, v_ref, tz_ref):
        zz = z_ref[...]
        c_ref[...] = zz @ p3_ref[...]
        v_ref[...] = zz @ w2_ref[...]
        tz_ref[...] = zz @ m3_ref[...] + o_ref[...]

    return pl.pallas_call(
        body,
        in_specs=[pl.BlockSpec((R, 16), lambda: (0, 0)),
                  pl.BlockSpec((16, 128), lambda: (0, 0)),
                  pl.BlockSpec((16, 128), lambda: (0, 0)),
                  pl.BlockSpec((16, AUXW), lambda: (0, 0)),
                  pl.BlockSpec((R, AUXW), lambda: (0, 0))],
        out_specs=[pl.BlockSpec((R, 128), lambda: (0, 0)),
                   pl.BlockSpec((R, 128), lambda: (0, 0)),
                   pl.BlockSpec((R, AUXW), lambda: (0, 0))],
        out_shape=[jax.ShapeDtypeStruct((R, 128), jnp.float32),
                   jax.ShapeDtypeStruct((R, 128), jnp.float32),
                   jax.ShapeDtypeStruct((R, AUXW), jnp.float32)],
    )(z, P3, Wc2, M3, ONE0)


def _sc_edge_pre(A, B, C, TZ, idx3, Ep, Np):
    epw = Ep // NW
    nblk = epw // BLK1
    nh = nblk // 2
    mesh = plsc.VectorSubcoreMesh(core_axis_name="c", subcore_axis_name="s")

    @functools.partial(
        pl.kernel, mesh=mesh,
        out_type=[jax.ShapeDtypeStruct((Ep, 128), jnp.float32),
                  jax.ShapeDtypeStruct((NC, Np, AUXW), jnp.float32)],
        compiler_params=_SC_PARAMS,
        scratch_types=[pltpu.VMEM((3, BLK1), jnp.int32)] * 2
        + [pltpu.VMEM((BLK1, 128), jnp.float32)] * 4
        + [pltpu.VMEM((BLK1, 128), jnp.float32),
           pltpu.VMEM((BLK1, 128), jnp.float32),
           pltpu.VMEM((BLK1, AUXW), jnp.float32),
           pltpu.VMEM_SHARED((64, 128), jnp.float32),
           pltpu.VMEM_SHARED((64, AUXW), jnp.float32),
           pltpu.VMEM_SHARED((Np, AUXW), jnp.float32),
           pltpu.SemaphoreType.DMA,
           pltpu.SemaphoreType.DMA],
    )
    def k(a_h, b_h, c_h, tz_h, idx_h, pre_h, aux_h,
          ib0, ib1, ab0, bb0, ab1, bb1, cb, pb, tb, cc, tzc,
          acc, sem0, sem1):
        cid = lax.axis_index("c")
        sid = lax.axis_index("s")
        w = sid * NC + cid
        base0 = pl.multiple_of(w * epw, 8)

        @pl.when(sid == 0)
        def _():
            pltpu.sync_copy(c_h, cc)
            pltpu.sync_copy(tz_h, tzc)

        @plsc.parallel_loop(0, BLK1)
        def _(i):
            tb[i, pl.ds(0, 16)] = jnp.zeros((16,), jnp.float32)
            tb[i, pl.ds(AUXW - 16, 16)] = jnp.zeros((16,), jnp.float32)

        rps = Np // NS
        for kk in range(rps // BLK1):
            pltpu.sync_copy(tb, acc.at[pl.ds(sid * rps + kk * BLK1, BLK1)])
        plsc.subcore_barrier()

        def fire(bi, ib, ab, bb, sem):
            pltpu.sync_copy(idx_h.at[w, bi], ib)
            pltpu.async_copy(a_h.at[ib.at[1]], ab, sem)
            pltpu.async_copy(b_h.at[ib.at[0]], bb, sem)

        def consume(bi, ib, ab, bb, sem):
            pltpu.sync_copy(cc.at[ib.at[2]], cb)
            pltpu.sync_copy(tzc.at[ib.at[2]], tb)
            pltpu.make_async_copy(a_h.at[ib.at[1]], ab, sem).wait()
            pltpu.make_async_copy(b_h.at[ib.at[0]], bb, sem).wait()

            @plsc.parallel_loop(0, BLK1, unroll=2)
            def _(e):
                for c in range(8):
                    s = pl.ds(c * 16, 16)
                    pb[e, s] = ab[e, s] + bb[e, s] + cb[e, s]

            pltpu.sync_copy(pb, pre_h.at[pl.ds(base0 + bi * BLK1, BLK1)])
            pltpu.sync_copy(tb, acc.at[ib.at[1]], add=True)

        fire(0, ib0, ab0, bb0, sem0)

        def body(g, carry):
            fire(2 * g + 1, ib1, ab1, bb1, sem1)
            consume(2 * g, ib0, ab0, bb0, sem0)

            @pl.when(g < nh - 1)
            def _():
                fire(2 * g + 2, ib0, ab0, bb0, sem0)

            consume(2 * g + 1, ib1, ab1, bb1, sem1)
            return carry

        lax.fori_loop(0, nh, body, 0)
        plsc.subcore_barrier()
        pltpu.sync_copy(acc.at[pl.ds(sid * rps, rps)],
                        aux_h.at[cid, pl.ds(sid * rps, rps)])

    return k(A, B, C, TZ, idx3)


def _tc_edge_exp(pre, y_hat, Ep):
    tb = 1024

    def body(pre_ref, y_ref, out_ref):
        s = _lr(pre_ref[...]) @ y_ref[...]
        e = jnp.exp(s)
        out_ref[...] = jnp.concatenate([e, jnp.zeros_like(e)], axis=1)

    return pl.pallas_call(
        body,
        grid=(Ep // tb,),
        in_specs=[pl.BlockSpec((tb, 128), lambda i: (i, 0)),
                  pl.BlockSpec((128, 8), lambda i: (0, 0))],
        out_specs=pl.BlockSpec((tb, 16), lambda i: (i, 0)),
        out_shape=jax.ShapeDtypeStruct((Ep, 16), jnp.float32),
    )(pre, y_hat)


def _sc_edge_agg(U, V, eE4, idx3, Ep, Np):
    epw = Ep // NW
    nblk = epw // BLK2
    nh = nblk // 2
    mesh = plsc.VectorSubcoreMesh(core_axis_name="c", subcore_axis_name="s")

    @functools.partial(
        pl.kernel, mesh=mesh,
        out_type=jax.ShapeDtypeStruct((NC, Np, ROWW), jnp.float32),
        compiler_params=_SC_PARAMS,
        scratch_types=[pltpu.VMEM((3, BLK2), jnp.int32)] * 2
        + [pltpu.VMEM((BLK2, 128), jnp.float32),
           pltpu.VMEM((BLK2, 16), jnp.float32)] * 2
        + [pltpu.VMEM((BLK2, 128), jnp.float32),
           pltpu.VMEM((BLK2, ROWW), jnp.float32),
           pltpu.VMEM_SHARED((64, 128), jnp.float32),
           pltpu.VMEM_SHARED((Np, ROWW), jnp.float32),
           pltpu.SemaphoreType.DMA,
           pltpu.SemaphoreType.DMA],
    )
    def k(u_h, v_h, e_h, idx_h, out_h,
          ib0, ib1, ub0, eb0, ub1, eb1, vb, mb, vc, acc, sem0, sem1):
        cid = lax.axis_index("c")
        sid = lax.axis_index("s")
        w = sid * NC + cid
        nz = ROWW // 16

        @pl.when(sid == 0)
        def _():
            pltpu.sync_copy(v_h, vc)

        @plsc.parallel_loop(0, BLK2 * nz)
        def _(i):
            mb[i // nz, pl.ds((i % nz) * 16, 16)] = jnp.zeros((16,),
                                                              jnp.float32)

        rps = Np // NS
        for kk in range(rps // BLK2):
            pltpu.sync_copy(mb, acc.at[pl.ds(sid * rps + kk * BLK2, BLK2)])
        plsc.subcore_barrier()

        def fire(bi, ib, ub, eb, sem):
            pltpu.sync_copy(idx_h.at[w, bi], ib)
            pltpu.async_copy(u_h.at[ib.at[0]], ub, sem)
            pltpu.async_copy(e_h.at[w, bi], eb, sem)

        def consume(bi, ib, ub, eb, sem):
            pltpu.sync_copy(vc.at[ib.at[2]], vb)
            pltpu.make_async_copy(u_h.at[ib.at[0]], ub, sem).wait()
            pltpu.make_async_copy(e_h.at[w, bi], eb, sem).wait()

            @plsc.parallel_loop(0, BLK2, unroll=2)
            def _(e):
                er = eb[e, :]
                for c in range(8):
                    s = pl.ds(c * 16, 16)
                    mb[e, s] = _lane_bcast(er, c) * (ub[e, s] + vb[e, s])
                mb[e, pl.ds(128, 16)] = er

            pltpu.sync_copy(mb, acc.at[ib.at[1]], add=True)

        fire(0, ib0, ub0, eb0, sem0)

        def body(g, carry):
            fire(2 * g + 1, ib1, ub1, eb1, sem1)
            consume(2 * g, ib0, ub0, eb0, sem0)

            @pl.when(g < nh - 1)
            def _():
                fire(2 * g + 2, ib0, ub0, eb0, sem0)

            consume(2 * g + 1, ib1, ub1, eb1, sem1)
            return carry

        lax.fori_loop(0, nh, body, 0)
        plsc.subcore_barrier()
        pltpu.sync_copy(acc.at[pl.ds(sid * rps, rps)],
                        out_h.at[cid, pl.ds(sid * rps, rps)])

    return k(U, V, eE4, idx3)


def _tc_final(A, B, U, h_p, parts, aux, P3, Wc2, y_hat, E8, res2, Np):
    def body(a_ref, b_ref, u_ref, h_ref, m0_ref, m1_ref, x0_ref, x1_ref,
             p3_ref, w2_ref, y_ref, e8_ref, r_ref, out_ref):
        x = x0_ref[...] + x1_ref[...]
        deg = x[:, 0:1]
        zbar = x[:, 1:17] / jnp.maximum(deg, 1.0)
        m = m0_ref[...] + m1_ref[...]
        pre = a_ref[...] + b_ref[...] + zbar @ p3_ref[...]
        es = jnp.exp(_lr(pre) @ y_ref[...])
        ts = u_ref[...] + zbar @ w2_ref[...]
        den = es + m[:, 128:136]
        e128 = es @ e8_ref[...]
        d128 = den @ e8_ref[...]
        agg = (e128 * ts + m[:, 0:128]) / d128
        hh = h_ref[...]
        hn = _lr(agg + r_ref[0, 0] * hh)
        out_ref[...] = jnp.where(deg > 0.0, hn, hh)

    g = Np // 256
    nspec = pl.BlockSpec((256, 128), lambda i: (i, 0))
    mspec = pl.BlockSpec((256, ROWW), lambda i: (i, 0))
    xspec = pl.BlockSpec((256, AUXW), lambda i: (i, 0))
    return pl.pallas_call(
        body,
        grid=(g,),
        in_specs=[nspec, nspec, nspec, nspec, mspec, mspec, xspec, xspec,
                  pl.BlockSpec((16, 128), lambda i: (0, 0)),
                  pl.BlockSpec((16, 128), lambda i: (0, 0)),
                  pl.BlockSpec((128, 8), lambda i: (0, 0)),
                  pl.BlockSpec((8, 128), lambda i: (0, 0)),
                  pl.BlockSpec(memory_space=pltpu.SMEM)],
        out_specs=nspec,
        out_shape=jax.ShapeDtypeStruct((Np, 128), jnp.float32),
    )(A, B, U, h_p, parts[0], parts[1], aux[0], aux[1],
      P3, Wc2, y_hat, E8, res2)


def kernel(h, z, edge_index, edge_type, Wc_w, P_hat_w, y_hat_w, res_w):
    N, D = h.shape
    R, RD = z.shape
    E = edge_type.shape[0]
    Np = ((N + 255) // 256) * 256
    gran = NW * BLK1 * 2
    Ep = ((E + gran - 1) // gran) * gran
    epw = Ep // NW

    src = edge_index[0].astype(jnp.int32)
    dst = edge_index[1].astype(jnp.int32)
    et = edge_type.astype(jnp.int32)
    srcp = jnp.full((Ep,), Np - 1, jnp.int32).at[:E].set(src)
    dstp = jnp.full((Ep,), Np - 1, jnp.int32).at[:E].set(dst)
    etp = jnp.zeros((Ep,), jnp.int32).at[:E].set(et)
    sde = jnp.stack([srcp.reshape(NW, epw), dstp.reshape(NW, epw),
                     etp.reshape(NW, epw)], axis=1)
    idx1 = jnp.swapaxes(sde.reshape(NW, 3, epw // BLK1, BLK1), 1, 2)
    idx2 = jnp.swapaxes(sde.reshape(NW, 3, epw // BLK2, BLK2), 1, 2)

    P1 = P_hat_w[:D]
    P2 = P_hat_w[D:2 * D]
    P3 = P_hat_w[2 * D:]
    Wc1 = Wc_w[:D]
    Wc2 = Wc_w[D:]

    M3 = jnp.concatenate([jnp.zeros((RD, 1), jnp.float32),
                          jnp.eye(RD, dtype=jnp.float32),
                          jnp.zeros((RD, AUXW - RD - 1), jnp.float32)],
                         axis=1)
    ONE0 = jnp.broadcast_to(
        (jnp.arange(AUXW) == 0).astype(jnp.float32)[None, :], (R, AUXW))
    E8 = jnp.kron(jnp.eye(8, dtype=jnp.float32),
                  jnp.ones((1, 16), jnp.float32))
    res2 = res_w.reshape(1, 1)

    h_p = jnp.pad(h, ((0, Np - N), (0, 0)))

    A, B, U = _tc_dense1(h_p, P1, P2, Wc1, Np)
    C, V, TZ = _tc_reltables(z, P3, Wc2, M3, ONE0, R)
    pre, aux = _sc_edge_pre(A, B, C, TZ, idx1, Ep, Np)
    eE = _tc_edge_exp(pre, y_hat_w, Ep)
    eE4 = eE.reshape(NW, epw // BLK2, BLK2, 16)
    parts = _sc_edge_agg(U, V, eE4, idx2, Ep, Np)
    out = _tc_final(A, B, U, h_p, parts, aux, P3, Wc2, y_hat_w, E8, res2, Np)
    return out[:N]


# speedup vs baseline: 24.5539x; 1.0032x over previous
import functools

import jax
import jax.numpy as jnp
from jax import lax
from jax.experimental import pallas as pl
from jax.experimental.pallas import tpu as pltpu
from jax.experimental.pallas import tpu_sc as plsc

NC, NS, LANES = 2, 16, 16
NW = NC * NS
BLK1 = 128
BLK2 = 64
ROWW = 144
AUXW = 24
_SC_PARAMS = pltpu.CompilerParams(use_tc_tiling_on_sc=False,
                                  needs_layout_passes=False)


def _lr(x):
    return jnp.maximum(x, 0.2 * x)


def _lane_bcast(v, lane):
    idx = jnp.full((LANES, 1), lane, jnp.int32)
    dn = lax.GatherDimensionNumbers(
        offset_dims=(), collapsed_slice_dims=(0,), start_index_map=(0,))
    return lax.gather(v, idx, dn, (1,),
                      mode=lax.GatherScatterMode.PROMISE_IN_BOUNDS)


def _tc_dense1(h_p, P1, P2, Wc1, z, P3, Wc2, M3, ONE0, Np, R):
    def body(h_ref, p1_ref, p2_ref, w1_ref, z_ref, p3_ref, w2_ref, m3_ref,
             o_ref, a_ref, b_ref, u_ref, c_ref, v_ref, tz_ref):
        hb = h_ref[...]
        a_ref[...] = hb @ p1_ref[...]
        b_ref[...] = hb @ p2_ref[...]
        u_ref[...] = hb @ w1_ref[...]

        @pl.when(pl.program_id(0) == 0)
        def _():
            zz = z_ref[...]
            c_ref[...] = zz @ p3_ref[...]
            v_ref[...] = zz @ w2_ref[...]
            tz_ref[...] = zz @ m3_ref[...] + o_ref[...]

    g = Np // 256
    wspec = pl.BlockSpec((128, 128), lambda i: (0, 0))
    nspec = pl.BlockSpec((256, 128), lambda i: (i, 0))
    zspec0 = pl.BlockSpec((R, 16), lambda i: (0, 0))
    w16 = pl.BlockSpec((16, 128), lambda i: (0, 0))
    wax = pl.BlockSpec((16, AUXW), lambda i: (0, 0))
    oax = pl.BlockSpec((R, AUXW), lambda i: (0, 0))
    return pl.pallas_call(
        body,
        grid=(g,),
        in_specs=[nspec, wspec, wspec, wspec, zspec0, w16, w16, wax, oax],
        out_specs=[nspec, nspec, nspec,
                   pl.BlockSpec((R, 128), lambda i: (0, 0)),
                   pl.BlockSpec((R, 128), lambda i: (0, 0)),
                   pl.BlockSpec((R, AUXW), lambda i: (0, 0))],
        out_shape=[jax.ShapeDtypeStruct((Np, 128), jnp.float32)] * 3
        + [jax.ShapeDtypeStruct((R, 128), jnp.float32),
           jax.ShapeDtypeStruct((R, 128), jnp.float32),
           jax.ShapeDtypeStruct((R, AUXW), jnp.float32)],
    )(h_p, P1, P2, Wc1, z, P3, Wc2, M3, ONE0)


def _sc_edge_pre(A, B, C, TZ, idx3, Ep, Np):
    epw = Ep // NW
    nblk = epw // BLK1
    nh = nblk // 2
    mesh = plsc.VectorSubcoreMesh(core_axis_name="c", subcore_axis_name="s")

    @functools.partial(
        pl.kernel, mesh=mesh,
        out_type=[jax.ShapeDtypeStruct((Ep, 128), jnp.float32),
                  jax.ShapeDtypeStruct((NC, Np, AUXW), jnp.float32)],
        compiler_params=_SC_PARAMS,
        scratch_types=[pltpu.VMEM((3, BLK1), jnp.int32)] * 2
        + [pltpu.VMEM((BLK1, 128), jnp.float32)] * 4
        + [pltpu.VMEM((BLK1, 128), jnp.float32),
           pltpu.VMEM((BLK1, 128), jnp.float32),
           pltpu.VMEM((BLK1, AUXW), jnp.float32),
           pltpu.VMEM_SHARED((64, 128), jnp.float32),
           pltpu.VMEM_SHARED((64, AUXW), jnp.float32),
           pltpu.VMEM_SHARED((Np, AUXW), jnp.float32),
           pltpu.SemaphoreType.DMA,
           pltpu.SemaphoreType.DMA],
    )
    def k(a_h, b_h, c_h, tz_h, idx_h, pre_h, aux_h,
          ib0, ib1, ab0, bb0, ab1, bb1, cb, pb, tb, cc, tzc,
          acc, sem0, sem1):
        cid = lax.axis_index("c")
        sid = lax.axis_index("s")
        w = sid * NC + cid
        base0 = pl.multiple_of(w * epw, 8)

        @pl.when(sid == 0)
        def _():
            pltpu.sync_copy(c_h, cc)
            pltpu.sync_copy(tz_h, tzc)

        @plsc.parallel_loop(0, BLK1)
        def _(i):
            tb[i, pl.ds(0, 16)] = jnp.zeros((16,), jnp.float32)
            tb[i, pl.ds(AUXW - 16, 16)] = jnp.zeros((16,), jnp.float32)

        rps = Np // NS
        for kk in range(rps // BLK1):
            pltpu.sync_copy(tb, acc.at[pl.ds(sid * rps + kk * BLK1, BLK1)])
        plsc.subcore_barrier()

        def fire(bi, ib, ab, bb, sem):
            pltpu.sync_copy(idx_h.at[w, bi], ib)
            pltpu.async_copy(a_h.at[ib.at[1]], ab, sem)
            pltpu.async_copy(b_h.at[ib.at[0]], bb, sem)

        def consume(bi, ib, ab, bb, sem):
            pltpu.sync_copy(cc.at[ib.at[2]], cb)
            pltpu.sync_copy(tzc.at[ib.at[2]], tb)
            pltpu.make_async_copy(a_h.at[ib.at[1]], ab, sem).wait()
            pltpu.make_async_copy(b_h.at[ib.at[0]], bb, sem).wait()

            @plsc.parallel_loop(0, BLK1, unroll=2)
            def _(e):
                for c in range(8):
                    s = pl.ds(c * 16, 16)
                    pb[e, s] = ab[e, s] + bb[e, s] + cb[e, s]

            pltpu.sync_copy(pb, pre_h.at[pl.ds(base0 + bi * BLK1, BLK1)])
            pltpu.sync_copy(tb, acc.at[ib.at[1]], add=True)

        fire(0, ib0, ab0, bb0, sem0)

        def body(g, carry):
            fire(2 * g + 1, ib1, ab1, bb1, sem1)
            consume(2 * g, ib0, ab0, bb0, sem0)

            @pl.when(g < nh - 1)
            def _():
                fire(2 * g + 2, ib0, ab0, bb0, sem0)

            consume(2 * g + 1, ib1, ab1, bb1, sem1)
            return carry

        lax.fori_loop(0, nh, body, 0)
        plsc.subcore_barrier()
        pltpu.sync_copy(acc.at[pl.ds(sid * rps, rps)],
                        aux_h.at[cid, pl.ds(sid * rps, rps)])

    return k(A, B, C, TZ, idx3)


def _tc_edge_exp(pre, y_hat, Ep):
    tb = 1024

    def body(pre_ref, y_ref, out_ref):
        s = _lr(pre_ref[...]) @ y_ref[...]
        e = jnp.exp(s)
        out_ref[...] = jnp.concatenate([e, jnp.zeros_like(e)], axis=1)

    return pl.pallas_call(
        body,
        grid=(Ep // tb,),
        in_specs=[pl.BlockSpec((tb, 128), lambda i: (i, 0)),
                  pl.BlockSpec((128, 8), lambda i: (0, 0))],
        out_specs=pl.BlockSpec((tb, 16), lambda i: (i, 0)),
        out_shape=jax.ShapeDtypeStruct((Ep, 16), jnp.float32),
    )(pre, y_hat)


def _sc_edge_agg(U, V, eE4, idx3, Ep, Np):
    epw = Ep // NW
    nblk = epw // BLK2
    nh = nblk // 2
    mesh = plsc.VectorSubcoreMesh(core_axis_name="c", subcore_axis_name="s")

    @functools.partial(
        pl.kernel, mesh=mesh,
        out_type=jax.ShapeDtypeStruct((NC, Np, ROWW), jnp.float32),
        compiler_params=_SC_PARAMS,
        scratch_types=[pltpu.VMEM((3, BLK2), jnp.int32)] * 2
        + [pltpu.VMEM((BLK2, 128), jnp.float32),
           pltpu.VMEM((BLK2, 16), jnp.float32)] * 2
        + [pltpu.VMEM((BLK2, 128), jnp.float32),
           pltpu.VMEM((BLK2, ROWW), jnp.float32),
           pltpu.VMEM_SHARED((64, 128), jnp.float32),
           pltpu.VMEM_SHARED((Np, ROWW), jnp.float32),
           pltpu.SemaphoreType.DMA,
           pltpu.SemaphoreType.DMA],
    )
    def k(u_h, v_h, e_h, idx_h, out_h,
          ib0, ib1, ub0, eb0, ub1, eb1, vb, mb, vc, acc, sem0, sem1):
        cid = lax.axis_index("c")
        sid = lax.axis_index("s")
        w = sid * NC + cid
        nz = ROWW // 16

        @pl.when(sid == 0)
        def _():
            pltpu.sync_copy(v_h, vc)

        @plsc.parallel_loop(0, BLK2 * nz)
        def _(i):
            mb[i // nz, pl.ds((i % nz) * 16, 16)] = jnp.zeros((16,),
                                                              jnp.float32)

        rps = Np // NS
        for kk in range(rps // BLK2):
            pltpu.sync_copy(mb, acc.at[pl.ds(sid * rps + kk * BLK2, BLK2)])
        plsc.subcore_barrier()

        def fire(bi, ib, ub, eb, sem):
            pltpu.sync_copy(idx_h.at[w, bi], ib)
            pltpu.async_copy(u_h.at[ib.at[0]], ub, sem)
            pltpu.async_copy(e_h.at[w, bi], eb, sem)

        def consume(bi, ib, ub, eb, sem):
            pltpu.sync_copy(vc.at[ib.at[2]], vb)
            pltpu.make_async_copy(u_h.at[ib.at[0]], ub, sem).wait()
            pltpu.make_async_copy(e_h.at[w, bi], eb, sem).wait()

            @plsc.parallel_loop(0, BLK2, unroll=2)
            def _(e):
                er = eb[e, :]
                for c in range(8):
                    s = pl.ds(c * 16, 16)
                    mb[e, s] = _lane_bcast(er, c) * (ub[e, s] + vb[e, s])
                mb[e, pl.ds(128, 16)] = er

            pltpu.sync_copy(mb, acc.at[ib.at[1]], add=True)

        fire(0, ib0, ub0, eb0, sem0)

        def body(g, carry):
            fire(2 * g + 1, ib1, ub1, eb1, sem1)
            consume(2 * g, ib0, ub0, eb0, sem0)

            @pl.when(g < nh - 1)
            def _():
                fire(2 * g + 2, ib0, ub0, eb0, sem0)

            consume(2 * g + 1, ib1, ub1, eb1, sem1)
            return carry

        lax.fori_loop(0, nh, body, 0)
        plsc.subcore_barrier()
        pltpu.sync_copy(acc.at[pl.ds(sid * rps, rps)],
                        out_h.at[cid, pl.ds(sid * rps, rps)])

    return k(U, V, eE4, idx3)


def _tc_final(A, B, U, h_p, parts, aux, P3, Wc2, y_hat, E8, res2, Np):
    def body(a_ref, b_ref, u_ref, h_ref, m0_ref, m1_ref, x0_ref, x1_ref,
             p3_ref, w2_ref, y_ref, e8_ref, r_ref, out_ref):
        x = x0_ref[...] + x1_ref[...]
        deg = x[:, 0:1]
        zbar = x[:, 1:17] / jnp.maximum(deg, 1.0)
        m = m0_ref[...] + m1_ref[...]
        pre = a_ref[...] + b_ref[...] + zbar @ p3_ref[...]
        es = jnp.exp(_lr(pre) @ y_ref[...])
        ts = u_ref[...] + zbar @ w2_ref[...]
        den = es + m[:, 128:136]
        e128 = es @ e8_ref[...]
        d128 = den @ e8_ref[...]
        agg = (e128 * ts + m[:, 0:128]) / d128
        hh = h_ref[...]
        hn = _lr(agg + r_ref[0, 0] * hh)
        out_ref[...] = jnp.where(deg > 0.0, hn, hh)

    g = Np // 256
    nspec = pl.BlockSpec((256, 128), lambda i: (i, 0))
    mspec = pl.BlockSpec((256, ROWW), lambda i: (i, 0))
    xspec = pl.BlockSpec((256, AUXW), lambda i: (i, 0))
    return pl.pallas_call(
        body,
        grid=(g,),
        in_specs=[nspec, nspec, nspec, nspec, mspec, mspec, xspec, xspec,
                  pl.BlockSpec((16, 128), lambda i: (0, 0)),
                  pl.BlockSpec((16, 128), lambda i: (0, 0)),
                  pl.BlockSpec((128, 8), lambda i: (0, 0)),
                  pl.BlockSpec((8, 128), lambda i: (0, 0)),
                  pl.BlockSpec(memory_space=pltpu.SMEM)],
        out_specs=nspec,
        out_shape=jax.ShapeDtypeStruct((Np, 128), jnp.float32),
    )(A, B, U, h_p, parts[0], parts[1], aux[0], aux[1],
      P3, Wc2, y_hat, E8, res2)


def kernel(h, z, edge_index, edge_type, Wc_w, P_hat_w, y_hat_w, res_w):
    N, D = h.shape
    R, RD = z.shape
    E = edge_type.shape[0]
    Np = ((N + 255) // 256) * 256
    gran = NW * BLK1 * 2
    Ep = ((E + gran - 1) // gran) * gran
    epw = Ep // NW

    src = edge_index[0].astype(jnp.int32)
    dst = edge_index[1].astype(jnp.int32)
    et = edge_type.astype(jnp.int32)
    srcp = jnp.full((Ep,), Np - 1, jnp.int32).at[:E].set(src)
    dstp = jnp.full((Ep,), Np - 1, jnp.int32).at[:E].set(dst)
    etp = jnp.zeros((Ep,), jnp.int32).at[:E].set(et)
    sde = jnp.stack([srcp.reshape(NW, epw), dstp.reshape(NW, epw),
                     etp.reshape(NW, epw)], axis=1)
    idx1 = jnp.swapaxes(sde.reshape(NW, 3, epw // BLK1, BLK1), 1, 2)
    idx2 = jnp.swapaxes(sde.reshape(NW, 3, epw // BLK2, BLK2), 1, 2)

    P1 = P_hat_w[:D]
    P2 = P_hat_w[D:2 * D]
    P3 = P_hat_w[2 * D:]
    Wc1 = Wc_w[:D]
    Wc2 = Wc_w[D:]

    M3 = jnp.concatenate([jnp.zeros((RD, 1), jnp.float32),
                          jnp.eye(RD, dtype=jnp.float32),
                          jnp.zeros((RD, AUXW - RD - 1), jnp.float32)],
                         axis=1)
    ONE0 = jnp.broadcast_to(
        (jnp.arange(AUXW) == 0).astype(jnp.float32)[None, :], (R, AUXW))
    E8 = jnp.kron(jnp.eye(8, dtype=jnp.float32),
                  jnp.ones((1, 16), jnp.float32))
    res2 = res_w.reshape(1, 1)

    h_p = jnp.pad(h, ((0, Np - N), (0, 0)))

    A, B, U, C, V, TZ = _tc_dense1(h_p, P1, P2, Wc1, z, P3, Wc2, M3,
                                    ONE0, Np, R)
    pre, aux = _sc_edge_pre(A, B, C, TZ, idx1, Ep, Np)
    eE = _tc_edge_exp(pre, y_hat_w, Ep)
    eE4 = eE.reshape(NW, epw // BLK2, BLK2, 16)
    parts = _sc_edge_agg(U, V, eE4, idx2, Ep, Np)
    out = _tc_final(A, B, U, h_p, parts, aux, P3, Wc2, y_hat_w, E8, res2, Np)
    return out[:N]


# speedup vs baseline: 26.8593x; 1.0939x over previous
import functools

import jax
import jax.numpy as jnp
from jax import lax
from jax.experimental import pallas as pl
from jax.experimental.pallas import tpu as pltpu
from jax.experimental.pallas import tpu_sc as plsc

NC, NS, LANES = 2, 16, 16
NW = NC * NS
BLK1 = 128
BLK2 = 64
ROWW = 144
AUXW = 24
_SC_PARAMS = pltpu.CompilerParams(use_tc_tiling_on_sc=False,
                                  needs_layout_passes=False)


def _lr(x):
    return jnp.maximum(x, 0.2 * x)


def _lane_bcast(v, lane):
    idx = jnp.full((LANES, 1), lane, jnp.int32)
    dn = lax.GatherDimensionNumbers(
        offset_dims=(), collapsed_slice_dims=(0,), start_index_map=(0,))
    return lax.gather(v, idx, dn, (1,),
                      mode=lax.GatherScatterMode.PROMISE_IN_BOUNDS)


def _tc_dense1(h_p, P1, P2, Wc1, z, P3, Wc2, M3, ONE0, Np, R):
    def body(h_ref, p1_ref, p2_ref, w1_ref, z_ref, p3_ref, w2_ref, m3_ref,
             o_ref, a_ref, b_ref, u_ref, c_ref, v_ref, tz_ref):
        hb = h_ref[...]
        a_ref[...] = hb @ p1_ref[...]
        b_ref[...] = hb @ p2_ref[...]
        u_ref[...] = hb @ w1_ref[...]

        @pl.when(pl.program_id(0) == 0)
        def _():
            zz = z_ref[...]
            c_ref[...] = zz @ p3_ref[...]
            v_ref[...] = zz @ w2_ref[...]
            tz_ref[...] = zz @ m3_ref[...] + o_ref[...]

    g = Np // 256
    wspec = pl.BlockSpec((128, 128), lambda i: (0, 0))
    nspec = pl.BlockSpec((256, 128), lambda i: (i, 0))
    zspec0 = pl.BlockSpec((R, 16), lambda i: (0, 0))
    w16 = pl.BlockSpec((16, 128), lambda i: (0, 0))
    wax = pl.BlockSpec((16, AUXW), lambda i: (0, 0))
    oax = pl.BlockSpec((R, AUXW), lambda i: (0, 0))
    return pl.pallas_call(
        body,
        grid=(g,),
        in_specs=[nspec, wspec, wspec, wspec, zspec0, w16, w16, wax, oax],
        out_specs=[nspec, nspec, nspec,
                   pl.BlockSpec((R, 128), lambda i: (0, 0)),
                   pl.BlockSpec((R, 128), lambda i: (0, 0)),
                   pl.BlockSpec((R, AUXW), lambda i: (0, 0))],
        out_shape=[jax.ShapeDtypeStruct((Np, 128), jnp.float32)] * 3
        + [jax.ShapeDtypeStruct((R, 128), jnp.float32),
           jax.ShapeDtypeStruct((R, 128), jnp.float32),
           jax.ShapeDtypeStruct((R, AUXW), jnp.float32)],
    )(h_p, P1, P2, Wc1, z, P3, Wc2, M3, ONE0)


def _sc_edge_pre(A, B, C, TZ, idx3, Ep, Np):
    epw = Ep // NW
    nblk = epw // BLK1
    nh = nblk // 2
    mesh = plsc.VectorSubcoreMesh(core_axis_name="c", subcore_axis_name="s")

    @functools.partial(
        pl.kernel, mesh=mesh,
        out_type=[jax.ShapeDtypeStruct((Ep, 128), jnp.float32),
                  jax.ShapeDtypeStruct((NC, Np, AUXW), jnp.float32)],
        compiler_params=_SC_PARAMS,
        scratch_types=[pltpu.VMEM((3, BLK1), jnp.int32)] * 2
        + [pltpu.VMEM((BLK1, 128), jnp.float32)] * 4
        + [pltpu.VMEM((BLK1, 128), jnp.float32),
           pltpu.VMEM((BLK1, 128), jnp.float32),
           pltpu.VMEM((BLK1, AUXW), jnp.float32),
           pltpu.VMEM_SHARED((64, 128), jnp.float32),
           pltpu.VMEM_SHARED((64, AUXW), jnp.float32),
           pltpu.VMEM_SHARED((Np, AUXW), jnp.float32),
           pltpu.SemaphoreType.DMA,
           pltpu.SemaphoreType.DMA],
    )
    def k(a_h, b_h, c_h, tz_h, idx_h, pre_h, aux_h,
          ib0, ib1, ab0, bb0, ab1, bb1, cb, pb, tb, cc, tzc,
          acc, sem0, sem1):
        cid = lax.axis_index("c")
        sid = lax.axis_index("s")
        w = sid * NC + cid
        base0 = pl.multiple_of(w * epw, 8)

        @pl.when(sid == 0)
        def _():
            pltpu.sync_copy(c_h, cc)
            pltpu.sync_copy(tz_h, tzc)

        @plsc.parallel_loop(0, BLK1)
        def _(i):
            tb[i, pl.ds(0, 16)] = jnp.zeros((16,), jnp.float32)
            tb[i, pl.ds(AUXW - 16, 16)] = jnp.zeros((16,), jnp.float32)

        rps = Np // NS
        for kk in range(rps // BLK1):
            pltpu.sync_copy(tb, acc.at[pl.ds(sid * rps + kk * BLK1, BLK1)])
        plsc.subcore_barrier()

        def fire(bi, ib, ab, bb, sem):
            pltpu.sync_copy(idx_h.at[w, bi], ib)
            pltpu.async_copy(a_h.at[ib.at[1]], ab, sem)
            pltpu.async_copy(b_h.at[ib.at[0]], bb, sem)

        def consume(bi, ib, ab, bb, sem):
            pltpu.sync_copy(cc.at[ib.at[2]], cb)
            pltpu.sync_copy(tzc.at[ib.at[2]], tb)
            pltpu.make_async_copy(a_h.at[ib.at[1]], ab, sem).wait()
            pltpu.make_async_copy(b_h.at[ib.at[0]], bb, sem).wait()

            @plsc.parallel_loop(0, BLK1, unroll=2)
            def _(e):
                for c in range(8):
                    s = pl.ds(c * 16, 16)
                    pb[e, s] = ab[e, s] + bb[e, s] + cb[e, s]

            pltpu.sync_copy(pb, pre_h.at[pl.ds(base0 + bi * BLK1, BLK1)])
            pltpu.sync_copy(tb, acc.at[ib.at[1]], add=True)

        fire(0, ib0, ab0, bb0, sem0)

        def body(g, carry):
            fire(2 * g + 1, ib1, ab1, bb1, sem1)
            consume(2 * g, ib0, ab0, bb0, sem0)

            @pl.when(g < nh - 1)
            def _():
                fire(2 * g + 2, ib0, ab0, bb0, sem0)

            consume(2 * g + 1, ib1, ab1, bb1, sem1)
            return carry

        lax.fori_loop(0, nh, body, 0)
        plsc.subcore_barrier()
        pltpu.sync_copy(acc.at[pl.ds(sid * rps, rps)],
                        aux_h.at[cid, pl.ds(sid * rps, rps)])

    return k(A, B, C, TZ, idx3)


def _tc_edge_exp(pre, y_hat, Ep):
    tb = 1024

    def body(pre_ref, y_ref, out_ref):
        s = _lr(pre_ref[...]) @ y_ref[...]
        e = jnp.exp(s)
        out_ref[...] = jnp.concatenate([e, jnp.zeros_like(e)], axis=1)

    return pl.pallas_call(
        body,
        grid=(Ep // tb,),
        in_specs=[pl.BlockSpec((tb, 128), lambda i: (i, 0)),
                  pl.BlockSpec((128, 8), lambda i: (0, 0))],
        out_specs=pl.BlockSpec((tb, 16), lambda i: (i, 0)),
        out_shape=jax.ShapeDtypeStruct((Ep, 16), jnp.float32),
    )(pre, y_hat)


def _sc_edge_agg(U, V, eE4, idx3, Ep, Np):
    epw = Ep // NW
    nblk = epw // BLK2
    nh = nblk // 2
    mesh = plsc.VectorSubcoreMesh(core_axis_name="c", subcore_axis_name="s")

    @functools.partial(
        pl.kernel, mesh=mesh,
        out_type=jax.ShapeDtypeStruct((NC, Np, ROWW), jnp.float32),
        compiler_params=_SC_PARAMS,
        scratch_types=[pltpu.VMEM((3, BLK2), jnp.int32)] * 2
        + [pltpu.VMEM((BLK2, 128), jnp.float32),
           pltpu.VMEM((BLK2, 16), jnp.float32)] * 2
        + [pltpu.VMEM((BLK2, 128), jnp.float32),
           pltpu.VMEM((BLK2, ROWW), jnp.float32),
           pltpu.VMEM_SHARED((64, 128), jnp.float32),
           pltpu.VMEM_SHARED((Np, ROWW), jnp.float32),
           pltpu.SemaphoreType.DMA,
           pltpu.SemaphoreType.DMA],
    )
    def k(u_h, v_h, e_h, idx_h, out_h,
          ib0, ib1, ub0, eb0, ub1, eb1, vb, mb, vc, acc, sem0, sem1):
        cid = lax.axis_index("c")
        sid = lax.axis_index("s")
        w = sid * NC + cid
        nz = ROWW // 16

        @pl.when(sid == 0)
        def _():
            pltpu.sync_copy(v_h, vc)

        @plsc.parallel_loop(0, BLK2 * nz)
        def _(i):
            mb[i // nz, pl.ds((i % nz) * 16, 16)] = jnp.zeros((16,),
                                                              jnp.float32)

        rps = Np // NS
        for kk in range(rps // BLK2):
            pltpu.sync_copy(mb, acc.at[pl.ds(sid * rps + kk * BLK2, BLK2)])
        plsc.subcore_barrier()

        def fire(bi, ib, ub, eb, sem):
            pltpu.sync_copy(idx_h.at[w, bi], ib)
            pltpu.async_copy(u_h.at[ib.at[0]], ub, sem)
            pltpu.async_copy(e_h.at[w, bi], eb, sem)

        def consume(bi, ib, ub, eb, sem):
            pltpu.sync_copy(vc.at[ib.at[2]], vb)
            pltpu.make_async_copy(u_h.at[ib.at[0]], ub, sem).wait()
            pltpu.make_async_copy(e_h.at[w, bi], eb, sem).wait()

            @plsc.parallel_loop(0, BLK2, unroll=2)
            def _(e):
                er = eb[e, :]
                for c in range(8):
                    s = pl.ds(c * 16, 16)
                    mb[e, s] = _lane_bcast(er, c) * (ub[e, s] + vb[e, s])
                mb[e, pl.ds(128, 16)] = er

            pltpu.sync_copy(mb, acc.at[ib.at[1]], add=True)

        fire(0, ib0, ub0, eb0, sem0)

        def body(g, carry):
            fire(2 * g + 1, ib1, ub1, eb1, sem1)
            consume(2 * g, ib0, ub0, eb0, sem0)

            @pl.when(g < nh - 1)
            def _():
                fire(2 * g + 2, ib0, ub0, eb0, sem0)

            consume(2 * g + 1, ib1, ub1, eb1, sem1)
            return carry

        lax.fori_loop(0, nh, body, 0)
        plsc.subcore_barrier()
        pltpu.sync_copy(acc.at[pl.ds(sid * rps, rps)],
                        out_h.at[cid, pl.ds(sid * rps, rps)])

    return k(U, V, eE4, idx3)


def _tc_final(A, B, U, h_p, parts, aux, P3, Wc2, y_hat, E8, res2, Np):
    def body(a_ref, b_ref, u_ref, h_ref, m0_ref, m1_ref, m2_ref, m3_ref,
             x0_ref, x1_ref, x2_ref, x3_ref,
             p3_ref, w2_ref, y_ref, e8_ref, r_ref, out_ref):
        x = (x0_ref[...] + x1_ref[...]) + (x2_ref[...] + x3_ref[...])
        deg = x[:, 0:1]
        zbar = x[:, 1:17] / jnp.maximum(deg, 1.0)
        m = (m0_ref[...] + m1_ref[...]) + (m2_ref[...] + m3_ref[...])
        pre = a_ref[...] + b_ref[...] + zbar @ p3_ref[...]
        es = jnp.exp(_lr(pre) @ y_ref[...])
        ts = u_ref[...] + zbar @ w2_ref[...]
        den = es + m[:, 128:136]
        e128 = es @ e8_ref[...]
        d128 = den @ e8_ref[...]
        agg = (e128 * ts + m[:, 0:128]) / d128
        hh = h_ref[...]
        hn = _lr(agg + r_ref[0, 0] * hh)
        out_ref[...] = jnp.where(deg > 0.0, hn, hh)

    g = Np // 256
    nspec = pl.BlockSpec((256, 128), lambda i: (i, 0))
    mspec = pl.BlockSpec((256, ROWW), lambda i: (i, 0))
    xspec = pl.BlockSpec((256, AUXW), lambda i: (i, 0))
    return pl.pallas_call(
        body,
        grid=(g,),
        in_specs=[nspec, nspec, nspec, nspec,
                  mspec, mspec, mspec, mspec, xspec, xspec, xspec, xspec,
                  pl.BlockSpec((16, 128), lambda i: (0, 0)),
                  pl.BlockSpec((16, 128), lambda i: (0, 0)),
                  pl.BlockSpec((128, 8), lambda i: (0, 0)),
                  pl.BlockSpec((8, 128), lambda i: (0, 0)),
                  pl.BlockSpec(memory_space=pltpu.SMEM)],
        out_specs=nspec,
        out_shape=jax.ShapeDtypeStruct((Np, 128), jnp.float32),
    )(A, B, U, h_p, *parts, *aux, P3, Wc2, y_hat, E8, res2)


def kernel(h, z, edge_index, edge_type, Wc_w, P_hat_w, y_hat_w, res_w):
    N, D = h.shape
    R, RD = z.shape
    E = edge_type.shape[0]
    Np = ((N + 255) // 256) * 256
    gran = NW * BLK1 * 4
    Ep = ((E + gran - 1) // gran) * gran
    epw = Ep // NW

    src = edge_index[0].astype(jnp.int32)
    dst = edge_index[1].astype(jnp.int32)
    et = edge_type.astype(jnp.int32)
    srcp = jnp.full((Ep,), Np - 1, jnp.int32).at[:E].set(src)
    dstp = jnp.full((Ep,), Np - 1, jnp.int32).at[:E].set(dst)
    etp = jnp.zeros((Ep,), jnp.int32).at[:E].set(et)
    sde = jnp.stack([srcp.reshape(NW, epw), dstp.reshape(NW, epw),
                     etp.reshape(NW, epw)], axis=1)
    eph = epw // 2
    sdeA, sdeB = sde[:, :, :eph], sde[:, :, eph:]
    idx1A = jnp.swapaxes(sdeA.reshape(NW, 3, eph // BLK1, BLK1), 1, 2)
    idx1B = jnp.swapaxes(sdeB.reshape(NW, 3, eph // BLK1, BLK1), 1, 2)
    idx2A = jnp.swapaxes(sdeA.reshape(NW, 3, eph // BLK2, BLK2), 1, 2)
    idx2B = jnp.swapaxes(sdeB.reshape(NW, 3, eph // BLK2, BLK2), 1, 2)

    P1 = P_hat_w[:D]
    P2 = P_hat_w[D:2 * D]
    P3 = P_hat_w[2 * D:]
    Wc1 = Wc_w[:D]
    Wc2 = Wc_w[D:]

    M3 = jnp.concatenate([jnp.zeros((RD, 1), jnp.float32),
                          jnp.eye(RD, dtype=jnp.float32),
                          jnp.zeros((RD, AUXW - RD - 1), jnp.float32)],
                         axis=1)
    ONE0 = jnp.broadcast_to(
        (jnp.arange(AUXW) == 0).astype(jnp.float32)[None, :], (R, AUXW))
    E8 = jnp.kron(jnp.eye(8, dtype=jnp.float32),
                  jnp.ones((1, 16), jnp.float32))
    res2 = res_w.reshape(1, 1)

    h_p = jnp.pad(h, ((0, Np - N), (0, 0)))

    A, B, U, C, V, TZ = _tc_dense1(h_p, P1, P2, Wc1, z, P3, Wc2, M3,
                                    ONE0, Np, R)
    Eh = Ep // 2
    preA, auxA = _sc_edge_pre(A, B, C, TZ, idx1A, Eh, Np)
    preB, auxB = _sc_edge_pre(A, B, C, TZ, idx1B, Eh, Np)
    eEA = _tc_edge_exp(preA, y_hat_w, Eh)
    eEB = _tc_edge_exp(preB, y_hat_w, Eh)
    eE4A = eEA.reshape(NW, eph // BLK2, BLK2, 16)
    eE4B = eEB.reshape(NW, eph // BLK2, BLK2, 16)
    partsA = _sc_edge_agg(U, V, eE4A, idx2A, Eh, Np)
    partsB = _sc_edge_agg(U, V, eE4B, idx2B, Eh, Np)
    out = _tc_final(A, B, U, h_p,
                    [partsA[0], partsA[1], partsB[0], partsB[1]],
                    [auxA[0], auxA[1], auxB[0], auxB[1]],
                    P3, Wc2, y_hat_w, E8, res2, Np)
    return out[:N]


# speedup vs baseline: 27.6647x; 1.0300x over previous
import functools

import jax
import jax.numpy as jnp
from jax import lax
from jax.experimental import pallas as pl
from jax.experimental.pallas import tpu as pltpu
from jax.experimental.pallas import tpu_sc as plsc

NC, NS, LANES = 2, 16, 16
NW = NC * NS
BLK1 = 128
BLK2 = 64
ROWW = 144
AUXW = 24
_SC_PARAMS = pltpu.CompilerParams(use_tc_tiling_on_sc=False,
                                  needs_layout_passes=False)


def _lr(x):
    return jnp.maximum(x, 0.2 * x)


def _lane_bcast(v, lane):
    idx = jnp.full((LANES, 1), lane, jnp.int32)
    dn = lax.GatherDimensionNumbers(
        offset_dims=(), collapsed_slice_dims=(0,), start_index_map=(0,))
    return lax.gather(v, idx, dn, (1,),
                      mode=lax.GatherScatterMode.PROMISE_IN_BOUNDS)


def _tc_dense1(h_p, P1, P2, Wc1, z, P3, Wc2, M3, ONE0, Np, R):
    def body(h_ref, p1_ref, p2_ref, w1_ref, z_ref, p3_ref, w2_ref, m3_ref,
             o_ref, a_ref, b_ref, u_ref, c_ref, v_ref, tz_ref):
        hb = h_ref[...]
        a_ref[...] = hb @ p1_ref[...]
        b_ref[...] = hb @ p2_ref[...]
        u_ref[...] = hb @ w1_ref[...]

        @pl.when(pl.program_id(0) == 0)
        def _():
            zz = z_ref[...]
            c_ref[...] = zz @ p3_ref[...]
            v_ref[...] = zz @ w2_ref[...]
            tz_ref[...] = zz @ m3_ref[...] + o_ref[...]

    g = Np // 256
    wspec = pl.BlockSpec((128, 128), lambda i: (0, 0))
    nspec = pl.BlockSpec((256, 128), lambda i: (i, 0))
    zspec0 = pl.BlockSpec((R, 16), lambda i: (0, 0))
    w16 = pl.BlockSpec((16, 128), lambda i: (0, 0))
    wax = pl.BlockSpec((16, AUXW), lambda i: (0, 0))
    oax = pl.BlockSpec((R, AUXW), lambda i: (0, 0))
    return pl.pallas_call(
        body,
        grid=(g,),
        in_specs=[nspec, wspec, wspec, wspec, zspec0, w16, w16, wax, oax],
        out_specs=[nspec, nspec, nspec,
                   pl.BlockSpec((R, 128), lambda i: (0, 0)),
                   pl.BlockSpec((R, 128), lambda i: (0, 0)),
                   pl.BlockSpec((R, AUXW), lambda i: (0, 0))],
        out_shape=[jax.ShapeDtypeStruct((Np, 128), jnp.float32)] * 3
        + [jax.ShapeDtypeStruct((R, 128), jnp.float32),
           jax.ShapeDtypeStruct((R, 128), jnp.float32),
           jax.ShapeDtypeStruct((R, AUXW), jnp.float32)],
    )(h_p, P1, P2, Wc1, z, P3, Wc2, M3, ONE0)


def _sc_edge_pre(A, B, C, TZ, idx3, eoff, Ep, Np):
    epw = Ep // NW
    nblk = epw // BLK1
    nh = nblk // 2
    mesh = plsc.VectorSubcoreMesh(core_axis_name="c", subcore_axis_name="s")

    @functools.partial(
        pl.kernel, mesh=mesh,
        out_type=[jax.ShapeDtypeStruct((Ep, 128), jnp.float32),
                  jax.ShapeDtypeStruct((NC, Np, AUXW), jnp.float32)],
        compiler_params=_SC_PARAMS,
        scratch_types=[pltpu.VMEM((3, BLK1), jnp.int32)] * 2
        + [pltpu.VMEM((BLK1, 128), jnp.float32)] * 4
        + [pltpu.VMEM((BLK1, 128), jnp.float32),
           pltpu.VMEM((BLK1, 128), jnp.float32),
           pltpu.VMEM((BLK1, AUXW), jnp.float32),
           pltpu.VMEM_SHARED((64, 128), jnp.float32),
           pltpu.VMEM_SHARED((64, AUXW), jnp.float32),
           pltpu.VMEM_SHARED((Np, AUXW), jnp.float32),
           pltpu.SemaphoreType.DMA,
           pltpu.SemaphoreType.DMA],
    )
    def k(a_h, b_h, c_h, tz_h, idx_h, pre_h, aux_h,
          ib0, ib1, ab0, bb0, ab1, bb1, cb, pb, tb, cc, tzc,
          acc, sem0, sem1):
        cid = lax.axis_index("c")
        sid = lax.axis_index("s")
        w = sid * NC + cid
        base0 = pl.multiple_of(w * epw, 8)

        @pl.when(sid == 0)
        def _():
            pltpu.sync_copy(c_h, cc)
            pltpu.sync_copy(tz_h, tzc)

        @plsc.parallel_loop(0, BLK1)
        def _(i):
            tb[i, pl.ds(0, 16)] = jnp.zeros((16,), jnp.float32)
            tb[i, pl.ds(AUXW - 16, 16)] = jnp.zeros((16,), jnp.float32)

        rps = Np // NS
        for kk in range(rps // BLK1):
            pltpu.sync_copy(tb, acc.at[pl.ds(sid * rps + kk * BLK1, BLK1)])
        plsc.subcore_barrier()

        def fire(bi, ib, ab, bb, sem):
            pltpu.sync_copy(
                idx_h.at[w, :, pl.ds(eoff + bi * BLK1, BLK1)], ib)
            pltpu.async_copy(a_h.at[ib.at[1]], ab, sem)
            pltpu.async_copy(b_h.at[ib.at[0]], bb, sem)

        def consume(bi, ib, ab, bb, sem):
            pltpu.sync_copy(cc.at[ib.at[2]], cb)
            pltpu.sync_copy(tzc.at[ib.at[2]], tb)
            pltpu.make_async_copy(a_h.at[ib.at[1]], ab, sem).wait()
            pltpu.make_async_copy(b_h.at[ib.at[0]], bb, sem).wait()

            @plsc.parallel_loop(0, BLK1, unroll=2)
            def _(e):
                for c in range(8):
                    s = pl.ds(c * 16, 16)
                    pb[e, s] = ab[e, s] + bb[e, s] + cb[e, s]

            pltpu.sync_copy(pb, pre_h.at[pl.ds(base0 + bi * BLK1, BLK1)])
            pltpu.sync_copy(tb, acc.at[ib.at[1]], add=True)

        fire(0, ib0, ab0, bb0, sem0)

        def body(g, carry):
            fire(2 * g + 1, ib1, ab1, bb1, sem1)
            consume(2 * g, ib0, ab0, bb0, sem0)

            @pl.when(g < nh - 1)
            def _():
                fire(2 * g + 2, ib0, ab0, bb0, sem0)

            consume(2 * g + 1, ib1, ab1, bb1, sem1)
            return carry

        lax.fori_loop(0, nh, body, 0)
        plsc.subcore_barrier()
        pltpu.sync_copy(acc.at[pl.ds(sid * rps, rps)],
                        aux_h.at[cid, pl.ds(sid * rps, rps)])

    return k(A, B, C, TZ, idx3)


def _tc_edge_exp(pre, y_hat, Ep):
    tb = 1024

    def body(pre_ref, y_ref, out_ref):
        s = _lr(pre_ref[...]) @ y_ref[...]
        e = jnp.exp(s)
        out_ref[...] = jnp.concatenate([e, jnp.zeros_like(e)], axis=1)

    return pl.pallas_call(
        body,
        grid=(Ep // tb,),
        in_specs=[pl.BlockSpec((tb, 128), lambda i: (i, 0)),
                  pl.BlockSpec((128, 8), lambda i: (0, 0))],
        out_specs=pl.BlockSpec((tb, 16), lambda i: (i, 0)),
        out_shape=jax.ShapeDtypeStruct((Ep, 16), jnp.float32),
    )(pre, y_hat)


def _sc_edge_agg(U, V, eE2, idx3, eoff, Ep, Np):
    epw = Ep // NW
    nblk = epw // BLK2
    nh = nblk // 2
    mesh = plsc.VectorSubcoreMesh(core_axis_name="c", subcore_axis_name="s")

    @functools.partial(
        pl.kernel, mesh=mesh,
        out_type=jax.ShapeDtypeStruct((NC, Np, ROWW), jnp.float32),
        compiler_params=_SC_PARAMS,
        scratch_types=[pltpu.VMEM((3, BLK2), jnp.int32)] * 2
        + [pltpu.VMEM((BLK2, 128), jnp.float32),
           pltpu.VMEM((BLK2, 16), jnp.float32)] * 2
        + [pltpu.VMEM((BLK2, 128), jnp.float32),
           pltpu.VMEM((BLK2, ROWW), jnp.float32),
           pltpu.VMEM_SHARED((64, 128), jnp.float32),
           pltpu.VMEM_SHARED((Np, ROWW), jnp.float32),
           pltpu.SemaphoreType.DMA,
           pltpu.SemaphoreType.DMA],
    )
    def k(u_h, v_h, e_h, idx_h, out_h,
          ib0, ib1, ub0, eb0, ub1, eb1, vb, mb, vc, acc, sem0, sem1):
        cid = lax.axis_index("c")
        sid = lax.axis_index("s")
        w = sid * NC + cid
        nz = ROWW // 16

        @pl.when(sid == 0)
        def _():
            pltpu.sync_copy(v_h, vc)

        @plsc.parallel_loop(0, BLK2 * nz)
        def _(i):
            mb[i // nz, pl.ds((i % nz) * 16, 16)] = jnp.zeros((16,),
                                                              jnp.float32)

        rps = Np // NS
        for kk in range(rps // BLK2):
            pltpu.sync_copy(mb, acc.at[pl.ds(sid * rps + kk * BLK2, BLK2)])
        plsc.subcore_barrier()

        base0e = pl.multiple_of(w * (Ep // NW), 8)

        def fire(bi, ib, ub, eb, sem):
            pltpu.sync_copy(
                idx_h.at[w, :, pl.ds(eoff + bi * BLK2, BLK2)], ib)
            pltpu.async_copy(u_h.at[ib.at[0]], ub, sem)
            pltpu.async_copy(e_h.at[pl.ds(base0e + bi * BLK2, BLK2)], eb,
                             sem)

        def consume(bi, ib, ub, eb, sem):
            pltpu.sync_copy(vc.at[ib.at[2]], vb)
            pltpu.make_async_copy(u_h.at[ib.at[0]], ub, sem).wait()
            pltpu.make_async_copy(
                e_h.at[pl.ds(base0e + bi * BLK2, BLK2)], eb, sem).wait()

            @plsc.parallel_loop(0, BLK2, unroll=2)
            def _(e):
                er = eb[e, :]
                for c in range(8):
                    s = pl.ds(c * 16, 16)
                    mb[e, s] = _lane_bcast(er, c) * (ub[e, s] + vb[e, s])
                mb[e, pl.ds(128, 16)] = er

            pltpu.sync_copy(mb, acc.at[ib.at[1]], add=True)

        fire(0, ib0, ub0, eb0, sem0)

        def body(g, carry):
            fire(2 * g + 1, ib1, ub1, eb1, sem1)
            consume(2 * g, ib0, ub0, eb0, sem0)

            @pl.when(g < nh - 1)
            def _():
                fire(2 * g + 2, ib0, ub0, eb0, sem0)

            consume(2 * g + 1, ib1, ub1, eb1, sem1)
            return carry

        lax.fori_loop(0, nh, body, 0)
        plsc.subcore_barrier()
        pltpu.sync_copy(acc.at[pl.ds(sid * rps, rps)],
                        out_h.at[cid, pl.ds(sid * rps, rps)])

    return k(U, V, eE2, idx3)


def _tc_final(A, B, U, h_p, parts, aux, P3, Wc2, y_hat, E8, res2, Np):
    def body(a_ref, b_ref, u_ref, h_ref, m0_ref, m1_ref, m2_ref, m3_ref,
             x0_ref, x1_ref, x2_ref, x3_ref,
             p3_ref, w2_ref, y_ref, e8_ref, r_ref, out_ref):
        x = (x0_ref[...] + x1_ref[...]) + (x2_ref[...] + x3_ref[...])
        deg = x[:, 0:1]
        zbar = x[:, 1:17] / jnp.maximum(deg, 1.0)
        m = (m0_ref[...] + m1_ref[...]) + (m2_ref[...] + m3_ref[...])
        pre = a_ref[...] + b_ref[...] + zbar @ p3_ref[...]
        es = jnp.exp(_lr(pre) @ y_ref[...])
        ts = u_ref[...] + zbar @ w2_ref[...]
        den = es + m[:, 128:136]
        e128 = es @ e8_ref[...]
        d128 = den @ e8_ref[...]
        agg = (e128 * ts + m[:, 0:128]) / d128
        hh = h_ref[...]
        hn = _lr(agg + r_ref[0, 0] * hh)
        out_ref[...] = jnp.where(deg > 0.0, hn, hh)

    g = Np // 256
    nspec = pl.BlockSpec((256, 128), lambda i: (i, 0))
    mspec = pl.BlockSpec((256, ROWW), lambda i: (i, 0))
    xspec = pl.BlockSpec((256, AUXW), lambda i: (i, 0))
    return pl.pallas_call(
        body,
        grid=(g,),
        in_specs=[nspec, nspec, nspec, nspec,
                  mspec, mspec, mspec, mspec, xspec, xspec, xspec, xspec,
                  pl.BlockSpec((16, 128), lambda i: (0, 0)),
                  pl.BlockSpec((16, 128), lambda i: (0, 0)),
                  pl.BlockSpec((128, 8), lambda i: (0, 0)),
                  pl.BlockSpec((8, 128), lambda i: (0, 0)),
                  pl.BlockSpec(memory_space=pltpu.SMEM)],
        out_specs=nspec,
        out_shape=jax.ShapeDtypeStruct((Np, 128), jnp.float32),
    )(A, B, U, h_p, *parts, *aux, P3, Wc2, y_hat, E8, res2)


def kernel(h, z, edge_index, edge_type, Wc_w, P_hat_w, y_hat_w, res_w):
    N, D = h.shape
    R, RD = z.shape
    E = edge_type.shape[0]
    Np = ((N + 255) // 256) * 256
    gran = NW * BLK1 * 4
    Ep = ((E + gran - 1) // gran) * gran
    epw = Ep // NW

    src = edge_index[0].astype(jnp.int32)
    dst = edge_index[1].astype(jnp.int32)
    et = edge_type.astype(jnp.int32)
    srcp = jnp.full((Ep,), Np - 1, jnp.int32).at[:E].set(src)
    dstp = jnp.full((Ep,), Np - 1, jnp.int32).at[:E].set(dst)
    etp = jnp.zeros((Ep,), jnp.int32).at[:E].set(et)
    sde = jnp.stack([srcp.reshape(NW, epw), dstp.reshape(NW, epw),
                     etp.reshape(NW, epw)], axis=1)
    eph = epw // 2

    P1 = P_hat_w[:D]
    P2 = P_hat_w[D:2 * D]
    P3 = P_hat_w[2 * D:]
    Wc1 = Wc_w[:D]
    Wc2 = Wc_w[D:]

    M3 = jnp.concatenate([jnp.zeros((RD, 1), jnp.float32),
                          jnp.eye(RD, dtype=jnp.float32),
                          jnp.zeros((RD, AUXW - RD - 1), jnp.float32)],
                         axis=1)
    ONE0 = jnp.broadcast_to(
        (jnp.arange(AUXW) == 0).astype(jnp.float32)[None, :], (R, AUXW))
    E8 = jnp.kron(jnp.eye(8, dtype=jnp.float32),
                  jnp.ones((1, 16), jnp.float32))
    res2 = res_w.reshape(1, 1)

    h_p = jnp.pad(h, ((0, Np - N), (0, 0)))

    A, B, U, C, V, TZ = _tc_dense1(h_p, P1, P2, Wc1, z, P3, Wc2, M3,
                                    ONE0, Np, R)
    Eh = Ep // 2
    preA, auxA = _sc_edge_pre(A, B, C, TZ, sde, 0, Eh, Np)
    preB, auxB = _sc_edge_pre(A, B, C, TZ, sde, eph, Eh, Np)
    eEA = _tc_edge_exp(preA, y_hat_w, Eh)
    eEB = _tc_edge_exp(preB, y_hat_w, Eh)
    partsA = _sc_edge_agg(U, V, eEA, sde, 0, Eh, Np)
    partsB = _sc_edge_agg(U, V, eEB, sde, eph, Eh, Np)
    out = _tc_final(A, B, U, h_p,
                    [partsA[0], partsA[1], partsB[0], partsB[1]],
                    [auxA[0], auxA[1], auxB[0], auxB[1]],
                    P3, Wc2, y_hat_w, E8, res2, Np)
    return out[:N]
